# baseline (speedup 1.0000x reference)
import functools

import jax
import jax.numpy as jnp
from jax import lax
from jax.experimental import pallas as pl
from jax.experimental.pallas import tpu as pltpu
from jax.experimental.pallas import tpu_sc as plsc

N = 10000
E = 160000
D = 128
G = 100
INV_AVG_ADJ = 1.0 / 16.0

NB_N = 10
BN = N // NB_N
NB_E = 100
BE = E // NB_E
CHUNK = 128
NCHUNKS = E // CHUNK
NWORKERS = 32
NSEG = 5
SEG_CHUNKS = NCHUNKS // NSEG
SEG_E = E // NSEG
SEG_BLOCKS = SEG_E // 1600
SEG_ITERS = -(-SEG_CHUNKS // NWORKERS)
ROWS_PER_TILE = 624
TAIL_ROWS = N - 16 * ROWS_PER_TILE
TAIL_BASE = 16 * ROWS_PER_TILE


def _silu(t):
    return t * jax.nn.sigmoid(t)


def _proj_body(x_ref, w1a_ref, w1b_ref, b1_ref, xa_ref, xb_ref):
    xblk = x_ref[...]
    xa_ref[...] = (
        jnp.dot(xblk, w1a_ref[...], preferred_element_type=jnp.float32)
        + b1_ref[...]
    )
    xb_ref[...] = jnp.dot(xblk, w1b_ref[...], preferred_element_type=jnp.float32)


def _proj(x, w1a, w1b, b1):
    bp = 2000
    return pl.pallas_call(
        _proj_body,
        grid=(N // bp,),
        in_specs=[
            pl.BlockSpec((bp, D), lambda i: (i, 0)),
            pl.BlockSpec((D, D), lambda i: (0, 0)),
            pl.BlockSpec((D, D), lambda i: (0, 0)),
            pl.BlockSpec((1, D), lambda i: (0, 0)),
        ],
        out_specs=[
            pl.BlockSpec((bp, D), lambda i: (i, 0)),
            pl.BlockSpec((bp, D), lambda i: (i, 0)),
        ],
        out_shape=[jax.ShapeDtypeStruct((N, D), jnp.float32)] * 2,
    )(x, w1a, w1b, b1)


def _sc_gather(xa, xb, idx2, seg):
    mesh = plsc.VectorSubcoreMesh(core_axis_name="c", subcore_axis_name="s")
    c0 = seg * SEG_CHUNKS
    c1 = c0 + SEG_CHUNKS

    @functools.partial(
        pl.kernel,
        out_type=jax.ShapeDtypeStruct((SEG_E, D), jnp.float32),
        mesh=mesh,
        scratch_types=[
            pltpu.VMEM((2, 2, CHUNK), jnp.int32),
            pltpu.VMEM((2, CHUNK, D), jnp.float32),
            pltpu.VMEM((2, CHUNK, D), jnp.float32),
        ] + [pltpu.SemaphoreType.DMA] * 8,
        name=f"sc_gather_{seg}",
    )
    def k(xa_hbm, xb_hbm, idx_hbm, g_hbm,
          idx_v, rows_a, rows_b,
          sem_i0, sem_i1, sem_a0, sem_a1, sem_b0, sem_b1, sem_w0, sem_w1):
        wid = lax.axis_index("s") * 2 + lax.axis_index("c")
        sem_i = [sem_i0, sem_i1]
        sem_a = [sem_a0, sem_a1]
        sem_b = [sem_b0, sem_b1]
        sem_w = [sem_w0, sem_w1]

        pltpu.async_copy(idx_hbm.at[c0 + wid], idx_v.at[0], sem_i0)

        def body(i, carry):
            slot = lax.rem(i, 2)
            nslot = lax.rem(i + 1, 2)
            c_prev = c0 + wid + NWORKERS * (i - 1)
            c_cur = c0 + wid + NWORKERS * i
            c_next = c0 + wid + NWORKERS * (i + 1)
            c_prev2 = c0 + wid + NWORKERS * (i - 2)

            @pl.when((i >= 1) & (c_prev < c1))
            def _():
                for s in range(2):
                    @pl.when(nslot == s)
                    def _():
                        pltpu.make_async_copy(
                            xa_hbm.at[idx_v.at[s, 0]], rows_a.at[s], sem_a[s]
                        ).wait()
                        pltpu.make_async_copy(
                            xb_hbm.at[idx_v.at[s, 1]], rows_b.at[s], sem_b[s]
                        ).wait()

            @pl.when(c_next < c1)
            def _():
                for s in range(2):
                    @pl.when(nslot == s)
                    def _():
                        pltpu.async_copy(idx_hbm.at[c_next], idx_v.at[s], sem_i[s])

            for s in range(2):
                @pl.when(slot == s)
                def _():
                    @pl.when((i >= 2) & (c_prev2 < c1))
                    def _():
                        pltpu.make_async_copy(
                            rows_a.at[s],
                            g_hbm.at[pl.ds((c_prev2 - c0) * CHUNK, CHUNK)],
                            sem_w[s],
                        ).wait()

                    @pl.when(c_cur < c1)
                    def _():
                        pltpu.make_async_copy(
                            idx_hbm.at[c_cur], idx_v.at[s], sem_i[s]
                        ).wait()
                        pltpu.async_copy(
                            xa_hbm.at[idx_v.at[s, 0]], rows_a.at[s], sem_a[s]
                        )
                        pltpu.async_copy(
                            xb_hbm.at[idx_v.at[s, 1]], rows_b.at[s], sem_b[s]
                        )

            @pl.when((i >= 1) & (c_prev < c1))
            def _():
                for s in range(2):
                    @pl.when(nslot == s)
                    def _():
                        def row_add(r, cc):
                            for jj in range(D // 16):
                                sl = pl.ds(jj * 16, 16)
                                rows_a[s, r, sl] = rows_a[s, r, sl] + rows_b[s, r, sl]
                            return cc

                        lax.fori_loop(0, CHUNK, row_add, 0)
                        pltpu.async_copy(
                            rows_a.at[s],
                            g_hbm.at[pl.ds((c_prev - c0) * CHUNK, CHUNK)],
                            sem_w[s],
                        )

            return carry

        lax.fori_loop(0, SEG_ITERS + 2, body, 0)

    return k(xa, xb, idx2)


def _edge_body(g_ref, ea_ref, w1c_ref, w2_ref, b2_ref, out_ref, half_ref):
    t = g_ref[...] + jnp.dot(
        ea_ref[...], w1c_ref[...], preferred_element_type=jnp.float32
    )
    h = _silu(t)
    en = (
        jnp.dot(h, w2_ref[...], preferred_element_type=jnp.float32)
        + b2_ref[...]
    )
    out_ref[...] = en
    half_ref[...] = en


def _edge_body_chain(g_ref, ea_ref, w1c_ref, w2_ref, b2_ref, prev_ref,
                     out_ref, half_ref):
    del prev_ref
    _edge_body(g_ref, ea_ref, w1c_ref, w2_ref, b2_ref, out_ref, half_ref)


def _edge_mlp(g, ea, w1c, w2, b2, seg, en_prev):
    blk = lambda i: (i + seg * SEG_BLOCKS, 0)
    loc = lambda i: (i, 0)
    whole = lambda i: (0, 0)
    in_specs = [
        pl.BlockSpec((BE, D), loc),
        pl.BlockSpec((BE, D), blk),
        pl.BlockSpec((D, D), whole),
        pl.BlockSpec((D, D), whole),
        pl.BlockSpec((1, D), whole),
    ]
    args = [g, ea, w1c, w2, b2]
    body = _edge_body
    aliases = {}
    if en_prev is not None:
        in_specs.append(pl.BlockSpec((8, D), whole))
        args.append(en_prev)
        body = _edge_body_chain
        aliases = {5: 0}
    return pl.pallas_call(
        body,
        grid=(SEG_BLOCKS,),
        in_specs=in_specs,
        out_specs=[
            pl.BlockSpec((BE, D), blk),
            pl.BlockSpec((BE, D), loc),
        ],
        out_shape=[
            jax.ShapeDtypeStruct((E, D), jnp.float32),
            jax.ShapeDtypeStruct((SEG_E, D), jnp.float32),
        ],
        input_output_aliases=aliases,
    )(*args)


def _sc_scatter(enh, dst2d, init, seg):
    mesh = plsc.VectorSubcoreMesh(core_axis_name="c", subcore_axis_name="s")
    c0 = seg * SEG_CHUNKS
    c1 = c0 + SEG_CHUNKS

    @functools.partial(
        pl.kernel,
        out_type=jax.ShapeDtypeStruct((2, N, D), jnp.float32),
        mesh=mesh,
        scratch_types=[
            pltpu.VMEM((2, CHUNK), jnp.int32),
            pltpu.VMEM((2, CHUNK, D), jnp.float32),
            pltpu.VMEM_SHARED((N, D), jnp.float32),
        ] + [pltpu.SemaphoreType.DMA] * 6,
        name=f"sc_scatter_{seg}",
    )
    def k(en_hbm, dst_hbm, init_hbm, out_hbm, idx_v, rows_v, acc_sh,
          sem_i0, sem_i1, sem_r0, sem_r1, sem_s0, sem_s1):
        cid = lax.axis_index("c")
        sid = lax.axis_index("s")
        wid = sid * 2 + cid
        r0 = pl.multiple_of(sid * ROWS_PER_TILE, 8)
        sem_i = [sem_i0, sem_i1]
        sem_r = [sem_r0, sem_r1]
        sem_s = [sem_s0, sem_s1]

        pltpu.async_copy(dst_hbm.at[c0 + wid], idx_v.at[0], sem_i0)
        pltpu.async_copy(en_hbm.at[pl.ds(wid * CHUNK, CHUNK)], rows_v.at[0], sem_r0)

        pltpu.sync_copy(
            init_hbm.at[cid, pl.ds(r0, ROWS_PER_TILE)],
            acc_sh.at[pl.ds(r0, ROWS_PER_TILE)],
        )

        @pl.when(sid == 0)
        def _():
            pltpu.sync_copy(
                init_hbm.at[cid, pl.ds(TAIL_BASE, TAIL_ROWS)],
                acc_sh.at[pl.ds(TAIL_BASE, TAIL_ROWS)],
            )

        plsc.subcore_barrier()

        def body(i, carry):
            slot = lax.rem(i, 2)
            nslot = lax.rem(i + 1, 2)
            c_prev = c0 + wid + NWORKERS * (i - 1)
            c_cur = c0 + wid + NWORKERS * i
            c_next = c0 + wid + NWORKERS * (i + 1)

            @pl.when((i >= 1) & (c_prev < c1))
            def _():
                for s in range(2):
                    @pl.when(nslot == s)
                    def _():
                        pltpu.make_async_copy(
                            rows_v.at[s], acc_sh.at[idx_v.at[s]], sem_s[s]
                        ).wait()

            @pl.when(c_next < c1)
            def _():
                for s in range(2):
                    @pl.when(nslot == s)
                    def _():
                        pltpu.async_copy(dst_hbm.at[c_next], idx_v.at[s], sem_i[s])
                        pltpu.async_copy(
                            en_hbm.at[pl.ds((c_next - c0) * CHUNK, CHUNK)],
                            rows_v.at[s], sem_r[s],
                        )

            @pl.when(c_cur < c1)
            def _():
                for s in range(2):
                    @pl.when(slot == s)
                    def _():
                        pltpu.make_async_copy(
                            dst_hbm.at[c_cur], idx_v.at[s], sem_i[s]
                        ).wait()
                        pltpu.make_async_copy(
                            en_hbm.at[pl.ds((c_cur - c0) * CHUNK, CHUNK)],
                            rows_v.at[s], sem_r[s],
                        ).wait()
                        pltpu.async_copy(
                            rows_v.at[s], acc_sh.at[idx_v.at[s]], sem_s[s],
                            add=True,
                        )

            return carry

        lax.fori_loop(0, SEG_ITERS + 1, body, 0)
        plsc.subcore_barrier()
        pltpu.sync_copy(
            acc_sh.at[pl.ds(r0, ROWS_PER_TILE)],
            out_hbm.at[cid, pl.ds(r0, ROWS_PER_TILE)],
        )

        @pl.when(sid == 0)
        def _():
            pltpu.sync_copy(
                acc_sh.at[pl.ds(TAIL_BASE, TAIL_ROWS)],
                out_hbm.at[cid, pl.ds(TAIL_BASE, TAIL_ROWS)],
            )

    return k(enh, dst2d, init)


def _node_body(x_ref, p0_ref, p1_ref, b_ref, upad_ref,
               n1a_ref, n1b_ref, n1c_ref, nb1_ref, nw2_ref, nb2_ref,
               g1a_ref, g1b_ref, gb1_ref, gw2_ref, gb2_ref,
               xn_ref, uout_ref, sums_ref, cnt_ref):
    i = pl.program_id(0)

    @pl.when(i == 0)
    def _():
        sums_ref[...] = jnp.zeros((D, D), jnp.float32)
        cnt_ref[...] = jnp.zeros((D, D), jnp.float32)

    oh = (b_ref[...] == lax.broadcasted_iota(jnp.int32, (BN, D), 1)).astype(
        jnp.float32
    )
    ug = jnp.dot(upad_ref[...], n1c_ref[...], preferred_element_type=jnp.float32)
    msgs = (p0_ref[...] + p1_ref[...]) * INV_AVG_ADJ
    pre = (
        jnp.dot(x_ref[...], n1a_ref[...], preferred_element_type=jnp.float32)
        + jnp.dot(msgs, n1b_ref[...], preferred_element_type=jnp.float32)
        + jnp.dot(oh, ug, preferred_element_type=jnp.float32)
        + nb1_ref[...]
    )
    xn = (
        jnp.dot(_silu(pre), nw2_ref[...], preferred_element_type=jnp.float32)
        + nb2_ref[...]
    )
    xn_ref[...] = xn

    dims = (((0,), (0,)), ((), ()))
    sums_ref[...] += lax.dot_general(
        oh, xn, dims, preferred_element_type=jnp.float32
    )
    cnt_ref[...] += lax.dot_general(
        oh, jnp.ones((BN, D), jnp.float32), dims, preferred_element_type=jnp.float32
    )

    @pl.when(i == NB_N - 1)
    def _():
        mean = sums_ref[...] / jnp.maximum(cnt_ref[...], 1.0)
        t = (
            jnp.dot(upad_ref[...], g1a_ref[...], preferred_element_type=jnp.float32)
            + jnp.dot(mean, g1b_ref[...], preferred_element_type=jnp.float32)
            + gb1_ref[...]
        )
        uout_ref[...] = (
            jnp.dot(_silu(t), gw2_ref[...], preferred_element_type=jnp.float32)
            + gb2_ref[...]
        )


def _node_global(x, p0, p1, batch2d, upad,
                 n1a, n1b, n1c, nb1, nw2, nb2,
                 g1a, g1b, gb1, gw2, gb2):
    whole = lambda i: (0, 0)
    blk = lambda i: (i, 0)
    return pl.pallas_call(
        _node_body,
        grid=(NB_N,),
        in_specs=[
            pl.BlockSpec((BN, D), blk),
            pl.BlockSpec((BN, D), blk),
            pl.BlockSpec((BN, D), blk),
            pl.BlockSpec((BN, 1), blk),
            pl.BlockSpec((D, D), whole),
            pl.BlockSpec((D, D), whole),
            pl.BlockSpec((D, D), whole),
            pl.BlockSpec((D, D), whole),
            pl.BlockSpec((1, D), whole),
            pl.BlockSpec((D, D), whole),
            pl.BlockSpec((1, D), whole),
            pl.BlockSpec((D, D), whole),
            pl.BlockSpec((D, D), whole),
            pl.BlockSpec((1, D), whole),
            pl.BlockSpec((D, D), whole),
            pl.BlockSpec((1, D), whole),
        ],
        out_specs=[
            pl.BlockSpec((BN, D), blk),
            pl.BlockSpec((D, D), whole),
        ],
        out_shape=[
            jax.ShapeDtypeStruct((N, D), jnp.float32),
            jax.ShapeDtypeStruct((D, D), jnp.float32),
        ],
        scratch_shapes=[
            pltpu.VMEM((D, D), jnp.float32),
            pltpu.VMEM((D, D), jnp.float32),
        ],
    )(x, p0, p1, batch2d, upad,
      n1a, n1b, n1c, nb1, nw2, nb2,
      g1a, g1b, gb1, gw2, gb2)


def kernel(x, edge_index, edge_attr, u, batch,
           e_w1, e_b1, e_w2, e_b2,
           n_w1, n_b1, n_w2, n_b2,
           g_w1, g_b1, g_w2, g_b2):
    src2d = edge_index[0].reshape(NCHUNKS, CHUNK)
    dst2d = edge_index[1].reshape(NCHUNKS, CHUNK)
    idx2 = jnp.stack([src2d, dst2d], axis=1)
    w1a, w1b, w1c = e_w1[:D], e_w1[D:2 * D], e_w1[2 * D:]
    n1a, n1b, n1c = n_w1[:D], n_w1[D:2 * D], n_w1[2 * D:]
    g1a, g1b = g_w1[:D], g_w1[D:]
    upad = jnp.zeros((D, D), jnp.float32).at[:G].set(u)
    batch2d = batch.reshape(N, 1)

    xa, xb = _proj(x, w1a, w1b, e_b1.reshape(1, D))
    e_b2r = e_b2.reshape(1, D)
    g_segs = [None] * NSEG
    g_segs[0] = _sc_gather(xa, xb, idx2, 0)
    en = None
    partials = jnp.zeros((2, N, D), jnp.float32)
    for seg in range(NSEG):
        if seg + 1 < NSEG:
            g_segs[seg + 1] = _sc_gather(xa, xb, idx2, seg + 1)
        en, enh = _edge_mlp(g_segs[seg], edge_attr, w1c, e_w2, e_b2r, seg, en)
        partials = _sc_scatter(enh, dst2d, partials, seg)
    x_new, uout = _node_global(
        x, partials[0], partials[1], batch2d, upad,
        n1a, n1b, n1c, n_b1.reshape(1, D), n_w2, n_b2.reshape(1, D),
        g1a, g1b, g_b1.reshape(1, D), g_w2, g_b2.reshape(1, D),
    )
    return (x_new, en, uout[:G])


# speedup vs baseline: 1.0652x; 1.0652x over previous
import functools

import jax
import jax.numpy as jnp
from jax import lax
from jax.experimental import pallas as pl
from jax.experimental.pallas import tpu as pltpu
from jax.experimental.pallas import tpu_sc as plsc

N = 10000
E = 160000
D = 128
G = 100
INV_AVG_ADJ = 1.0 / 16.0

NB_N = 10
BN = N // NB_N
NB_E = 100
BE = E // NB_E
CHUNK = 128
NCHUNKS = E // CHUNK
NWORKERS = 32
NSEG = 1
SEG_CHUNKS = NCHUNKS // NSEG
SEG_E = E // NSEG
SEG_BLOCKS = SEG_E // 1600
SEG_ITERS = -(-SEG_CHUNKS // NWORKERS)
ROWS_PER_TILE = 624
TAIL_ROWS = N - 16 * ROWS_PER_TILE
TAIL_BASE = 16 * ROWS_PER_TILE


def _silu(t):
    return t * jax.nn.sigmoid(t)


def _proj_body(x_ref, w1a_ref, w1b_ref, b1_ref, xa_ref, xb_ref):
    xblk = x_ref[...]
    xa_ref[...] = (
        jnp.dot(xblk, w1a_ref[...], preferred_element_type=jnp.float32)
        + b1_ref[...]
    )
    xb_ref[...] = jnp.dot(xblk, w1b_ref[...], preferred_element_type=jnp.float32)


def _proj(x, w1a, w1b, b1):
    bp = 2000
    return pl.pallas_call(
        _proj_body,
        grid=(N // bp,),
        in_specs=[
            pl.BlockSpec((bp, D), lambda i: (i, 0)),
            pl.BlockSpec((D, D), lambda i: (0, 0)),
            pl.BlockSpec((D, D), lambda i: (0, 0)),
            pl.BlockSpec((1, D), lambda i: (0, 0)),
        ],
        out_specs=[
            pl.BlockSpec((bp, D), lambda i: (i, 0)),
            pl.BlockSpec((bp, D), lambda i: (i, 0)),
        ],
        out_shape=[jax.ShapeDtypeStruct((N, D), jnp.float32)] * 2,
    )(x, w1a, w1b, b1)


def _sc_gather(xa, xb, idx2, seg):
    mesh = plsc.VectorSubcoreMesh(core_axis_name="c", subcore_axis_name="s")
    c0 = seg * SEG_CHUNKS
    c1 = c0 + SEG_CHUNKS

    @functools.partial(
        pl.kernel,
        out_type=jax.ShapeDtypeStruct((SEG_E, D), jnp.float32),
        mesh=mesh,
        scratch_types=[
            pltpu.VMEM((2, 2, CHUNK), jnp.int32),
            pltpu.VMEM((2, CHUNK, D), jnp.float32),
            pltpu.VMEM((2, CHUNK, D), jnp.float32),
        ] + [pltpu.SemaphoreType.DMA] * 8,
        name=f"sc_gather_{seg}",
    )
    def k(xa_hbm, xb_hbm, idx_hbm, g_hbm,
          idx_v, rows_a, rows_b,
          sem_i0, sem_i1, sem_a0, sem_a1, sem_b0, sem_b1, sem_w0, sem_w1):
        wid = lax.axis_index("s") * 2 + lax.axis_index("c")
        sem_i = [sem_i0, sem_i1]
        sem_a = [sem_a0, sem_a1]
        sem_b = [sem_b0, sem_b1]
        sem_w = [sem_w0, sem_w1]

        pltpu.async_copy(idx_hbm.at[c0 + wid], idx_v.at[0], sem_i0)

        def body(i, carry):
            slot = lax.rem(i, 2)
            nslot = lax.rem(i + 1, 2)
            c_prev = c0 + wid + NWORKERS * (i - 1)
            c_cur = c0 + wid + NWORKERS * i
            c_next = c0 + wid + NWORKERS * (i + 1)
            c_prev2 = c0 + wid + NWORKERS * (i - 2)

            @pl.when((i >= 1) & (c_prev < c1))
            def _():
                for s in range(2):
                    @pl.when(nslot == s)
                    def _():
                        pltpu.make_async_copy(
                            xa_hbm.at[idx_v.at[s, 0]], rows_a.at[s], sem_a[s]
                        ).wait()
                        pltpu.make_async_copy(
                            xb_hbm.at[idx_v.at[s, 1]], rows_b.at[s], sem_b[s]
                        ).wait()

            @pl.when(c_next < c1)
            def _():
                for s in range(2):
                    @pl.when(nslot == s)
                    def _():
                        pltpu.async_copy(idx_hbm.at[c_next], idx_v.at[s], sem_i[s])

            for s in range(2):
                @pl.when(slot == s)
                def _():
                    @pl.when((i >= 2) & (c_prev2 < c1))
                    def _():
                        pltpu.make_async_copy(
                            rows_a.at[s],
                            g_hbm.at[pl.ds((c_prev2 - c0) * CHUNK, CHUNK)],
                            sem_w[s],
                        ).wait()

                    @pl.when(c_cur < c1)
                    def _():
                        pltpu.make_async_copy(
                            idx_hbm.at[c_cur], idx_v.at[s], sem_i[s]
                        ).wait()
                        pltpu.async_copy(
                            xa_hbm.at[idx_v.at[s, 0]], rows_a.at[s], sem_a[s]
                        )
                        pltpu.async_copy(
                            xb_hbm.at[idx_v.at[s, 1]], rows_b.at[s], sem_b[s]
                        )

            @pl.when((i >= 1) & (c_prev < c1))
            def _():
                for s in range(2):
                    @pl.when(nslot == s)
                    def _():
                        def row_add(r, cc):
                            for jj in range(D // 16):
                                sl = pl.ds(jj * 16, 16)
                                rows_a[s, r, sl] = rows_a[s, r, sl] + rows_b[s, r, sl]
                            return cc

                        lax.fori_loop(0, CHUNK, row_add, 0)
                        pltpu.async_copy(
                            rows_a.at[s],
                            g_hbm.at[pl.ds((c_prev - c0) * CHUNK, CHUNK)],
                            sem_w[s],
                        )

            return carry

        lax.fori_loop(0, SEG_ITERS + 2, body, 0)

    return k(xa, xb, idx2)


def _edge_body(g_ref, ea_ref, w1c_ref, w2_ref, b2_ref, out_ref):
    t = g_ref[...] + jnp.dot(
        ea_ref[...], w1c_ref[...], preferred_element_type=jnp.float32
    )
    h = _silu(t)
    out_ref[...] = (
        jnp.dot(h, w2_ref[...], preferred_element_type=jnp.float32)
        + b2_ref[...]
    )


def _edge_mlp(g, ea, w1c, w2, b2):
    blk = lambda i: (i, 0)
    whole = lambda i: (0, 0)
    return pl.pallas_call(
        _edge_body,
        grid=(NB_E,),
        in_specs=[
            pl.BlockSpec((BE, D), blk),
            pl.BlockSpec((BE, D), blk),
            pl.BlockSpec((D, D), whole),
            pl.BlockSpec((D, D), whole),
            pl.BlockSpec((1, D), whole),
        ],
        out_specs=pl.BlockSpec((BE, D), blk),
        out_shape=jax.ShapeDtypeStruct((E, D), jnp.float32),
    )(g, ea, w1c, w2, b2)


def _sc_scatter(enh, dst2d, init, seg):
    mesh = plsc.VectorSubcoreMesh(core_axis_name="c", subcore_axis_name="s")
    c0 = seg * SEG_CHUNKS
    c1 = c0 + SEG_CHUNKS

    @functools.partial(
        pl.kernel,
        out_type=jax.ShapeDtypeStruct((2, N, D), jnp.float32),
        mesh=mesh,
        scratch_types=[
            pltpu.VMEM((2, CHUNK), jnp.int32),
            pltpu.VMEM((2, CHUNK, D), jnp.float32),
            pltpu.VMEM_SHARED((N, D), jnp.float32),
        ] + [pltpu.SemaphoreType.DMA] * 6,
        name=f"sc_scatter_{seg}",
    )
    def k(en_hbm, dst_hbm, init_hbm, out_hbm, idx_v, rows_v, acc_sh,
          sem_i0, sem_i1, sem_r0, sem_r1, sem_s0, sem_s1):
        cid = lax.axis_index("c")
        sid = lax.axis_index("s")
        wid = sid * 2 + cid
        r0 = pl.multiple_of(sid * ROWS_PER_TILE, 8)
        sem_i = [sem_i0, sem_i1]
        sem_r = [sem_r0, sem_r1]
        sem_s = [sem_s0, sem_s1]

        pltpu.async_copy(dst_hbm.at[c0 + wid], idx_v.at[0], sem_i0)
        pltpu.async_copy(en_hbm.at[pl.ds(wid * CHUNK, CHUNK)], rows_v.at[0], sem_r0)

        pltpu.sync_copy(
            init_hbm.at[cid, pl.ds(r0, ROWS_PER_TILE)],
            acc_sh.at[pl.ds(r0, ROWS_PER_TILE)],
        )

        @pl.when(sid == 0)
        def _():
            pltpu.sync_copy(
                init_hbm.at[cid, pl.ds(TAIL_BASE, TAIL_ROWS)],
                acc_sh.at[pl.ds(TAIL_BASE, TAIL_ROWS)],
            )

        plsc.subcore_barrier()

        def body(i, carry):
            slot = lax.rem(i, 2)
            nslot = lax.rem(i + 1, 2)
            c_prev = c0 + wid + NWORKERS * (i - 1)
            c_cur = c0 + wid + NWORKERS * i
            c_next = c0 + wid + NWORKERS * (i + 1)

            @pl.when((i >= 1) & (c_prev < c1))
            def _():
                for s in range(2):
                    @pl.when(nslot == s)
                    def _():
                        pltpu.make_async_copy(
                            rows_v.at[s], acc_sh.at[idx_v.at[s]], sem_s[s]
                        ).wait()

            @pl.when(c_next < c1)
            def _():
                for s in range(2):
                    @pl.when(nslot == s)
                    def _():
                        pltpu.async_copy(dst_hbm.at[c_next], idx_v.at[s], sem_i[s])
                        pltpu.async_copy(
                            en_hbm.at[pl.ds((c_next - c0) * CHUNK, CHUNK)],
                            rows_v.at[s], sem_r[s],
                        )

            @pl.when(c_cur < c1)
            def _():
                for s in range(2):
                    @pl.when(slot == s)
                    def _():
                        pltpu.make_async_copy(
                            dst_hbm.at[c_cur], idx_v.at[s], sem_i[s]
                        ).wait()
                        pltpu.make_async_copy(
                            en_hbm.at[pl.ds((c_cur - c0) * CHUNK, CHUNK)],
                            rows_v.at[s], sem_r[s],
                        ).wait()
                        pltpu.async_copy(
                            rows_v.at[s], acc_sh.at[idx_v.at[s]], sem_s[s],
                            add=True,
                        )

            return carry

        lax.fori_loop(0, SEG_ITERS + 1, body, 0)
        plsc.subcore_barrier()
        pltpu.sync_copy(
            acc_sh.at[pl.ds(r0, ROWS_PER_TILE)],
            out_hbm.at[cid, pl.ds(r0, ROWS_PER_TILE)],
        )

        @pl.when(sid == 0)
        def _():
            pltpu.sync_copy(
                acc_sh.at[pl.ds(TAIL_BASE, TAIL_ROWS)],
                out_hbm.at[cid, pl.ds(TAIL_BASE, TAIL_ROWS)],
            )

    return k(enh, dst2d, init)


def _node_body(x_ref, p0_ref, p1_ref, b_ref, upad_ref,
               n1a_ref, n1b_ref, n1c_ref, nb1_ref, nw2_ref, nb2_ref,
               g1a_ref, g1b_ref, gb1_ref, gw2_ref, gb2_ref,
               xn_ref, uout_ref, sums_ref, cnt_ref):
    i = pl.program_id(0)

    @pl.when(i == 0)
    def _():
        sums_ref[...] = jnp.zeros((D, D), jnp.float32)
        cnt_ref[...] = jnp.zeros((D, D), jnp.float32)

    oh = (b_ref[...] == lax.broadcasted_iota(jnp.int32, (BN, D), 1)).astype(
        jnp.float32
    )
    ug = jnp.dot(upad_ref[...], n1c_ref[...], preferred_element_type=jnp.float32)
    msgs = (p0_ref[...] + p1_ref[...]) * INV_AVG_ADJ
    pre = (
        jnp.dot(x_ref[...], n1a_ref[...], preferred_element_type=jnp.float32)
        + jnp.dot(msgs, n1b_ref[...], preferred_element_type=jnp.float32)
        + jnp.dot(oh, ug, preferred_element_type=jnp.float32)
        + nb1_ref[...]
    )
    xn = (
        jnp.dot(_silu(pre), nw2_ref[...], preferred_element_type=jnp.float32)
        + nb2_ref[...]
    )
    xn_ref[...] = xn

    dims = (((0,), (0,)), ((), ()))
    sums_ref[...] += lax.dot_general(
        oh, xn, dims, preferred_element_type=jnp.float32
    )
    cnt_ref[...] += lax.dot_general(
        oh, jnp.ones((BN, D), jnp.float32), dims, preferred_element_type=jnp.float32
    )

    @pl.when(i == NB_N - 1)
    def _():
        mean = sums_ref[...] / jnp.maximum(cnt_ref[...], 1.0)
        t = (
            jnp.dot(upad_ref[...], g1a_ref[...], preferred_element_type=jnp.float32)
            + jnp.dot(mean, g1b_ref[...], preferred_element_type=jnp.float32)
            + gb1_ref[...]
        )
        uout_ref[...] = (
            jnp.dot(_silu(t), gw2_ref[...], preferred_element_type=jnp.float32)
            + gb2_ref[...]
        )


def _node_global(x, p0, p1, batch2d, upad,
                 n1a, n1b, n1c, nb1, nw2, nb2,
                 g1a, g1b, gb1, gw2, gb2):
    whole = lambda i: (0, 0)
    blk = lambda i: (i, 0)
    return pl.pallas_call(
        _node_body,
        grid=(NB_N,),
        in_specs=[
            pl.BlockSpec((BN, D), blk),
            pl.BlockSpec((BN, D), blk),
            pl.BlockSpec((BN, D), blk),
            pl.BlockSpec((BN, 1), blk),
            pl.BlockSpec((D, D), whole),
            pl.BlockSpec((D, D), whole),
            pl.BlockSpec((D, D), whole),
            pl.BlockSpec((D, D), whole),
            pl.BlockSpec((1, D), whole),
            pl.BlockSpec((D, D), whole),
            pl.BlockSpec((1, D), whole),
            pl.BlockSpec((D, D), whole),
            pl.BlockSpec((D, D), whole),
            pl.BlockSpec((1, D), whole),
            pl.BlockSpec((D, D), whole),
            pl.BlockSpec((1, D), whole),
        ],
        out_specs=[
            pl.BlockSpec((BN, D), blk),
            pl.BlockSpec((D, D), whole),
        ],
        out_shape=[
            jax.ShapeDtypeStruct((N, D), jnp.float32),
            jax.ShapeDtypeStruct((D, D), jnp.float32),
        ],
        scratch_shapes=[
            pltpu.VMEM((D, D), jnp.float32),
            pltpu.VMEM((D, D), jnp.float32),
        ],
    )(x, p0, p1, batch2d, upad,
      n1a, n1b, n1c, nb1, nw2, nb2,
      g1a, g1b, gb1, gw2, gb2)


def kernel(x, edge_index, edge_attr, u, batch,
           e_w1, e_b1, e_w2, e_b2,
           n_w1, n_b1, n_w2, n_b2,
           g_w1, g_b1, g_w2, g_b2):
    src2d = edge_index[0].reshape(NCHUNKS, CHUNK)
    dst2d = edge_index[1].reshape(NCHUNKS, CHUNK)
    idx2 = jnp.stack([src2d, dst2d], axis=1)
    w1a, w1b, w1c = e_w1[:D], e_w1[D:2 * D], e_w1[2 * D:]
    n1a, n1b, n1c = n_w1[:D], n_w1[D:2 * D], n_w1[2 * D:]
    g1a, g1b = g_w1[:D], g_w1[D:]
    upad = jnp.zeros((D, D), jnp.float32).at[:G].set(u)
    batch2d = batch.reshape(N, 1)

    xa, xb = _proj(x, w1a, w1b, e_b1.reshape(1, D))
    g = _sc_gather(xa, xb, idx2, 0)
    en = _edge_mlp(g, edge_attr, w1c, e_w2, e_b2.reshape(1, D))
    zeros = jnp.zeros((2, N, D), jnp.float32)
    partials = _sc_scatter(en, dst2d, zeros, 0)
    x_new, uout = _node_global(
        x, partials[0], partials[1], batch2d, upad,
        n1a, n1b, n1c, n_b1.reshape(1, D), n_w2, n_b2.reshape(1, D),
        g1a, g1b, g_b1.reshape(1, D), g_w2, g_b2.reshape(1, D),
    )
    return (x_new, en, uout[:G])


# speedup vs baseline: 1.1871x; 1.1144x over previous
import functools

import jax
import jax.numpy as jnp
from jax import lax
from jax.experimental import pallas as pl
from jax.experimental.pallas import tpu as pltpu
from jax.experimental.pallas import tpu_sc as plsc

N = 10000
E = 160000
D = 128
G = 100
INV_AVG_ADJ = 1.0 / 16.0

NB_N = 10
BN = N // NB_N
NB_E = 50
BE = E // NB_E
CHUNK = 128
NCHUNKS = E // CHUNK
NWORKERS = 32
NSEG = 1
SEG_CHUNKS = NCHUNKS // NSEG
SEG_E = E // NSEG
SEG_BLOCKS = SEG_E // 1600
SEG_ITERS = -(-SEG_CHUNKS // NWORKERS)
ROWS_PER_TILE = 624
TAIL_ROWS = N - 16 * ROWS_PER_TILE
TAIL_BASE = 16 * ROWS_PER_TILE


def _silu(t):
    return t * jax.nn.sigmoid(t)


def _proj_body(x_ref, w1a_ref, w1b_ref, b1_ref, xa_ref, xb_ref):
    xblk = x_ref[...]
    xa_ref[...] = (
        jnp.dot(xblk, w1a_ref[...], preferred_element_type=jnp.float32)
        + b1_ref[...]
    )
    xb_ref[...] = jnp.dot(xblk, w1b_ref[...], preferred_element_type=jnp.float32)


def _proj(x, w1a, w1b, b1):
    bp = 2000
    return pl.pallas_call(
        _proj_body,
        grid=(N // bp,),
        in_specs=[
            pl.BlockSpec((bp, D), lambda i: (i, 0)),
            pl.BlockSpec((D, D), lambda i: (0, 0)),
            pl.BlockSpec((D, D), lambda i: (0, 0)),
            pl.BlockSpec((1, D), lambda i: (0, 0)),
        ],
        out_specs=[
            pl.BlockSpec((bp, D), lambda i: (i, 0)),
            pl.BlockSpec((bp, D), lambda i: (i, 0)),
        ],
        out_shape=[jax.ShapeDtypeStruct((N, D), jnp.float32)] * 2,
    )(x, w1a, w1b, b1)


def _sc_gather(xa, xb, idx2, seg):
    mesh = plsc.VectorSubcoreMesh(core_axis_name="c", subcore_axis_name="s")
    c0 = seg * SEG_CHUNKS
    c1 = c0 + SEG_CHUNKS

    @functools.partial(
        pl.kernel,
        out_type=jax.ShapeDtypeStruct((SEG_E, D), jnp.float32),
        mesh=mesh,
        scratch_types=[
            pltpu.VMEM((2, 2, CHUNK), jnp.int32),
            pltpu.VMEM((2, CHUNK, D), jnp.float32),
            pltpu.VMEM((2, CHUNK, D), jnp.float32),
        ] + [pltpu.SemaphoreType.DMA] * 8,
        name=f"sc_gather_{seg}",
    )
    def k(xa_hbm, xb_hbm, idx_hbm, g_hbm,
          idx_v, rows_a, rows_b,
          sem_i0, sem_i1, sem_a0, sem_a1, sem_b0, sem_b1, sem_w0, sem_w1):
        wid = lax.axis_index("s") * 2 + lax.axis_index("c")
        sem_i = [sem_i0, sem_i1]
        sem_a = [sem_a0, sem_a1]
        sem_b = [sem_b0, sem_b1]
        sem_w = [sem_w0, sem_w1]

        pltpu.async_copy(idx_hbm.at[c0 + wid], idx_v.at[0], sem_i0)

        def body(i, carry):
            slot = lax.rem(i, 2)
            nslot = lax.rem(i + 1, 2)
            c_prev = c0 + wid + NWORKERS * (i - 1)
            c_cur = c0 + wid + NWORKERS * i
            c_next = c0 + wid + NWORKERS * (i + 1)
            c_prev2 = c0 + wid + NWORKERS * (i - 2)

            @pl.when((i >= 1) & (c_prev < c1))
            def _():
                for s in range(2):
                    @pl.when(nslot == s)
                    def _():
                        pltpu.make_async_copy(
                            xa_hbm.at[idx_v.at[s, 0]], rows_a.at[s], sem_a[s]
                        ).wait()
                        pltpu.make_async_copy(
                            xb_hbm.at[idx_v.at[s, 1]], rows_b.at[s], sem_b[s]
                        ).wait()

            @pl.when(c_next < c1)
            def _():
                for s in range(2):
                    @pl.when(nslot == s)
                    def _():
                        pltpu.async_copy(idx_hbm.at[c_next], idx_v.at[s], sem_i[s])

            for s in range(2):
                @pl.when(slot == s)
                def _():
                    @pl.when((i >= 2) & (c_prev2 < c1))
                    def _():
                        pltpu.make_async_copy(
                            rows_a.at[s],
                            g_hbm.at[pl.ds((c_prev2 - c0) * CHUNK, CHUNK)],
                            sem_w[s],
                        ).wait()

                    @pl.when(c_cur < c1)
                    def _():
                        pltpu.make_async_copy(
                            idx_hbm.at[c_cur], idx_v.at[s], sem_i[s]
                        ).wait()
                        pltpu.async_copy(
                            xa_hbm.at[idx_v.at[s, 0]], rows_a.at[s], sem_a[s]
                        )
                        pltpu.async_copy(
                            xb_hbm.at[idx_v.at[s, 1]], rows_b.at[s], sem_b[s]
                        )

            @pl.when((i >= 1) & (c_prev < c1))
            def _():
                for s in range(2):
                    @pl.when(nslot == s)
                    def _():
                        def row_add(r, cc):
                            for jj in range(D // 16):
                                sl = pl.ds(jj * 16, 16)
                                rows_a[s, r, sl] = rows_a[s, r, sl] + rows_b[s, r, sl]
                            return cc

                        lax.fori_loop(0, CHUNK, row_add, 0)
                        pltpu.async_copy(
                            rows_a.at[s],
                            g_hbm.at[pl.ds((c_prev - c0) * CHUNK, CHUNK)],
                            sem_w[s],
                        )

            return carry

        lax.fori_loop(0, SEG_ITERS + 2, body, 0)

    return k(xa, xb, idx2)


def _edge_body(g_ref, ea_ref, w1c_ref, w2_ref, b2_ref, out_ref):
    t = g_ref[...] + jnp.dot(
        ea_ref[...], w1c_ref[...], preferred_element_type=jnp.float32
    )
    h = _silu(t)
    out_ref[...] = (
        jnp.dot(h, w2_ref[...], preferred_element_type=jnp.float32)
        + b2_ref[...]
    )


def _edge_mlp(g, ea, w1c, w2, b2):
    blk = lambda i: (i, 0)
    whole = lambda i: (0, 0)
    return pl.pallas_call(
        _edge_body,
        grid=(NB_E,),
        in_specs=[
            pl.BlockSpec((BE, D), blk),
            pl.BlockSpec((BE, D), blk),
            pl.BlockSpec((D, D), whole),
            pl.BlockSpec((D, D), whole),
            pl.BlockSpec((1, D), whole),
        ],
        out_specs=pl.BlockSpec((BE, D), blk),
        out_shape=jax.ShapeDtypeStruct((E, D), jnp.float32),
    )(g, ea, w1c, w2, b2)


def _sc_scatter(enh, dst2d, init, seg):
    mesh = plsc.VectorSubcoreMesh(core_axis_name="c", subcore_axis_name="s")
    c0 = seg * SEG_CHUNKS
    c1 = c0 + SEG_CHUNKS

    @functools.partial(
        pl.kernel,
        out_type=jax.ShapeDtypeStruct((2, N, D), jnp.float32),
        mesh=mesh,
        scratch_types=[
            pltpu.VMEM((2, CHUNK), jnp.int32),
            pltpu.VMEM((2, CHUNK, D), jnp.float32),
            pltpu.VMEM_SHARED((N, D), jnp.float32),
        ] + [pltpu.SemaphoreType.DMA] * 6,
        name=f"sc_scatter_{seg}",
    )
    def k(en_hbm, dst_hbm, init_hbm, out_hbm, idx_v, rows_v, acc_sh,
          sem_i0, sem_i1, sem_r0, sem_r1, sem_s0, sem_s1):
        cid = lax.axis_index("c")
        sid = lax.axis_index("s")
        wid = sid * 2 + cid
        r0 = pl.multiple_of(sid * ROWS_PER_TILE, 8)
        sem_i = [sem_i0, sem_i1]
        sem_r = [sem_r0, sem_r1]
        sem_s = [sem_s0, sem_s1]

        pltpu.async_copy(dst_hbm.at[c0 + wid], idx_v.at[0], sem_i0)
        pltpu.async_copy(en_hbm.at[pl.ds(wid * CHUNK, CHUNK)], rows_v.at[0], sem_r0)

        pltpu.sync_copy(
            init_hbm.at[cid, pl.ds(r0, ROWS_PER_TILE)],
            acc_sh.at[pl.ds(r0, ROWS_PER_TILE)],
        )

        @pl.when(sid == 0)
        def _():
            pltpu.sync_copy(
                init_hbm.at[cid, pl.ds(TAIL_BASE, TAIL_ROWS)],
                acc_sh.at[pl.ds(TAIL_BASE, TAIL_ROWS)],
            )

        plsc.subcore_barrier()

        def body(i, carry):
            slot = lax.rem(i, 2)
            nslot = lax.rem(i + 1, 2)
            c_prev = c0 + wid + NWORKERS * (i - 1)
            c_cur = c0 + wid + NWORKERS * i
            c_next = c0 + wid + NWORKERS * (i + 1)

            @pl.when((i >= 1) & (c_prev < c1))
            def _():
                for s in range(2):
                    @pl.when(nslot == s)
                    def _():
                        pltpu.make_async_copy(
                            rows_v.at[s], acc_sh.at[idx_v.at[s]], sem_s[s]
                        ).wait()

            @pl.when(c_next < c1)
            def _():
                for s in range(2):
                    @pl.when(nslot == s)
                    def _():
                        pltpu.async_copy(dst_hbm.at[c_next], idx_v.at[s], sem_i[s])
                        pltpu.async_copy(
                            en_hbm.at[pl.ds((c_next - c0) * CHUNK, CHUNK)],
                            rows_v.at[s], sem_r[s],
                        )

            @pl.when(c_cur < c1)
            def _():
                for s in range(2):
                    @pl.when(slot == s)
                    def _():
                        pltpu.make_async_copy(
                            dst_hbm.at[c_cur], idx_v.at[s], sem_i[s]
                        ).wait()
                        pltpu.make_async_copy(
                            en_hbm.at[pl.ds((c_cur - c0) * CHUNK, CHUNK)],
                            rows_v.at[s], sem_r[s],
                        ).wait()
                        pltpu.async_copy(
                            rows_v.at[s], acc_sh.at[idx_v.at[s]], sem_s[s],
                            add=True,
                        )

            return carry

        lax.fori_loop(0, SEG_ITERS + 1, body, 0)
        plsc.subcore_barrier()
        pltpu.sync_copy(
            acc_sh.at[pl.ds(r0, ROWS_PER_TILE)],
            out_hbm.at[cid, pl.ds(r0, ROWS_PER_TILE)],
        )

        @pl.when(sid == 0)
        def _():
            pltpu.sync_copy(
                acc_sh.at[pl.ds(TAIL_BASE, TAIL_ROWS)],
                out_hbm.at[cid, pl.ds(TAIL_BASE, TAIL_ROWS)],
            )

    return k(enh, dst2d, init)


def _node_body(x_ref, p0_ref, p1_ref, b_ref, upad_ref,
               n1a_ref, n1b_ref, n1c_ref, nb1_ref, nw2_ref, nb2_ref,
               g1a_ref, g1b_ref, gb1_ref, gw2_ref, gb2_ref,
               xn_ref, uout_ref, sums_ref, cnt_ref):
    i = pl.program_id(0)

    @pl.when(i == 0)
    def _():
        sums_ref[...] = jnp.zeros((D, D), jnp.float32)
        cnt_ref[...] = jnp.zeros((D, D), jnp.float32)

    oh = (b_ref[...] == lax.broadcasted_iota(jnp.int32, (BN, D), 1)).astype(
        jnp.float32
    )
    ug = jnp.dot(upad_ref[...], n1c_ref[...], preferred_element_type=jnp.float32)
    msgs = (p0_ref[...] + p1_ref[...]) * INV_AVG_ADJ
    pre = (
        jnp.dot(x_ref[...], n1a_ref[...], preferred_element_type=jnp.float32)
        + jnp.dot(msgs, n1b_ref[...], preferred_element_type=jnp.float32)
        + jnp.dot(oh, ug, preferred_element_type=jnp.float32)
        + nb1_ref[...]
    )
    xn = (
        jnp.dot(_silu(pre), nw2_ref[...], preferred_element_type=jnp.float32)
        + nb2_ref[...]
    )
    xn_ref[...] = xn

    dims = (((0,), (0,)), ((), ()))
    sums_ref[...] += lax.dot_general(
        oh, xn, dims, preferred_element_type=jnp.float32
    )
    cnt_ref[...] += lax.dot_general(
        oh, jnp.ones((BN, D), jnp.float32), dims, preferred_element_type=jnp.float32
    )

    @pl.when(i == NB_N - 1)
    def _():
        mean = sums_ref[...] / jnp.maximum(cnt_ref[...], 1.0)
        t = (
            jnp.dot(upad_ref[...], g1a_ref[...], preferred_element_type=jnp.float32)
            + jnp.dot(mean, g1b_ref[...], preferred_element_type=jnp.float32)
            + gb1_ref[...]
        )
        uout_ref[...] = (
            jnp.dot(_silu(t), gw2_ref[...], preferred_element_type=jnp.float32)
            + gb2_ref[...]
        )


def _node_global(x, p0, p1, batch2d, upad,
                 n1a, n1b, n1c, nb1, nw2, nb2,
                 g1a, g1b, gb1, gw2, gb2):
    whole = lambda i: (0, 0)
    blk = lambda i: (i, 0)
    return pl.pallas_call(
        _node_body,
        grid=(NB_N,),
        in_specs=[
            pl.BlockSpec((BN, D), blk),
            pl.BlockSpec((BN, D), blk),
            pl.BlockSpec((BN, D), blk),
            pl.BlockSpec((BN, 1), blk),
            pl.BlockSpec((D, D), whole),
            pl.BlockSpec((D, D), whole),
            pl.BlockSpec((D, D), whole),
            pl.BlockSpec((D, D), whole),
            pl.BlockSpec((1, D), whole),
            pl.BlockSpec((D, D), whole),
            pl.BlockSpec((1, D), whole),
            pl.BlockSpec((D, D), whole),
            pl.BlockSpec((D, D), whole),
            pl.BlockSpec((1, D), whole),
            pl.BlockSpec((D, D), whole),
            pl.BlockSpec((1, D), whole),
        ],
        out_specs=[
            pl.BlockSpec((BN, D), blk),
            pl.BlockSpec((D, D), whole),
        ],
        out_shape=[
            jax.ShapeDtypeStruct((N, D), jnp.float32),
            jax.ShapeDtypeStruct((D, D), jnp.float32),
        ],
        scratch_shapes=[
            pltpu.VMEM((D, D), jnp.float32),
            pltpu.VMEM((D, D), jnp.float32),
        ],
    )(x, p0, p1, batch2d, upad,
      n1a, n1b, n1c, nb1, nw2, nb2,
      g1a, g1b, gb1, gw2, gb2)


def kernel(x, edge_index, edge_attr, u, batch,
           e_w1, e_b1, e_w2, e_b2,
           n_w1, n_b1, n_w2, n_b2,
           g_w1, g_b1, g_w2, g_b2):
    src2d = edge_index[0].reshape(NCHUNKS, CHUNK)
    dst2d = edge_index[1].reshape(NCHUNKS, CHUNK)
    idx2 = jnp.stack([src2d, dst2d], axis=1)
    w1a, w1b, w1c = e_w1[:D], e_w1[D:2 * D], e_w1[2 * D:]
    n1a, n1b, n1c = n_w1[:D], n_w1[D:2 * D], n_w1[2 * D:]
    g1a, g1b = g_w1[:D], g_w1[D:]
    upad = jnp.zeros((D, D), jnp.float32).at[:G].set(u)
    batch2d = batch.reshape(N, 1)

    xa, xb = _proj(x, w1a, w1b, e_b1.reshape(1, D))
    g = _sc_gather(xa, xb, idx2, 0)
    en = _edge_mlp(g, edge_attr, w1c, e_w2, e_b2.reshape(1, D))
    zeros = jnp.zeros((2, N, D), jnp.float32)
    partials = _sc_scatter(en, dst2d, zeros, 0)
    x_new, uout = _node_global(
        x, partials[0], partials[1], batch2d, upad,
        n1a, n1b, n1c, n_b1.reshape(1, D), n_w2, n_b2.reshape(1, D),
        g1a, g1b, g_b1.reshape(1, D), g_w2, g_b2.reshape(1, D),
    )
    return (x_new, en, uout[:G])


# speedup vs baseline: 1.2277x; 1.0342x over previous
import functools

import jax
import jax.numpy as jnp
from jax import lax
from jax.experimental import pallas as pl
from jax.experimental.pallas import tpu as pltpu
from jax.experimental.pallas import tpu_sc as plsc

N = 10000
E = 160000
D = 128
G = 100
INV_AVG_ADJ = 1.0 / 16.0

NB_N = 10
BN = N // NB_N
NB_E = 25
BE = E // NB_E
CHUNK = 128
NCHUNKS = E // CHUNK
NWORKERS = 32
NSEG = 1
SEG_CHUNKS = NCHUNKS // NSEG
SEG_E = E // NSEG
SEG_BLOCKS = SEG_E // 1600
SEG_ITERS = -(-SEG_CHUNKS // NWORKERS)
ROWS_PER_TILE = 624
TAIL_ROWS = N - 16 * ROWS_PER_TILE
TAIL_BASE = 16 * ROWS_PER_TILE


def _silu(t):
    return t * jax.nn.sigmoid(t)


def _proj_body(x_ref, w1a_ref, w1b_ref, b1_ref, xa_ref, xb_ref):
    xblk = x_ref[...]
    xa_ref[...] = (
        jnp.dot(xblk, w1a_ref[...], preferred_element_type=jnp.float32)
        + b1_ref[...]
    )
    xb_ref[...] = jnp.dot(xblk, w1b_ref[...], preferred_element_type=jnp.float32)


def _proj(x, w1a, w1b, b1):
    bp = 2000
    return pl.pallas_call(
        _proj_body,
        grid=(N // bp,),
        in_specs=[
            pl.BlockSpec((bp, D), lambda i: (i, 0)),
            pl.BlockSpec((D, D), lambda i: (0, 0)),
            pl.BlockSpec((D, D), lambda i: (0, 0)),
            pl.BlockSpec((1, D), lambda i: (0, 0)),
        ],
        out_specs=[
            pl.BlockSpec((bp, D), lambda i: (i, 0)),
            pl.BlockSpec((bp, D), lambda i: (i, 0)),
        ],
        out_shape=[jax.ShapeDtypeStruct((N, D), jnp.float32)] * 2,
    )(x, w1a, w1b, b1)


def _sc_gather(xa, xb, idx2, seg):
    mesh = plsc.VectorSubcoreMesh(core_axis_name="c", subcore_axis_name="s")
    c0 = seg * SEG_CHUNKS
    c1 = c0 + SEG_CHUNKS

    @functools.partial(
        pl.kernel,
        out_type=jax.ShapeDtypeStruct((SEG_E, D), jnp.float32),
        mesh=mesh,
        scratch_types=[
            pltpu.VMEM((2, 2, CHUNK), jnp.int32),
            pltpu.VMEM((2, CHUNK, D), jnp.float32),
            pltpu.VMEM((2, CHUNK, D), jnp.float32),
        ] + [pltpu.SemaphoreType.DMA] * 8,
        name=f"sc_gather_{seg}",
    )
    def k(xa_hbm, xb_hbm, idx_hbm, g_hbm,
          idx_v, rows_a, rows_b,
          sem_i0, sem_i1, sem_a0, sem_a1, sem_b0, sem_b1, sem_w0, sem_w1):
        wid = lax.axis_index("s") * 2 + lax.axis_index("c")
        sem_i = [sem_i0, sem_i1]
        sem_a = [sem_a0, sem_a1]
        sem_b = [sem_b0, sem_b1]
        sem_w = [sem_w0, sem_w1]

        pltpu.async_copy(idx_hbm.at[c0 + wid], idx_v.at[0], sem_i0)

        def body(i, carry):
            slot = lax.rem(i, 2)
            nslot = lax.rem(i + 1, 2)
            c_prev = c0 + wid + NWORKERS * (i - 1)
            c_cur = c0 + wid + NWORKERS * i
            c_next = c0 + wid + NWORKERS * (i + 1)
            c_prev2 = c0 + wid + NWORKERS * (i - 2)

            @pl.when((i >= 1) & (c_prev < c1))
            def _():
                for s in range(2):
                    @pl.when(nslot == s)
                    def _():
                        pltpu.make_async_copy(
                            xa_hbm.at[idx_v.at[s, 0]], rows_a.at[s], sem_a[s]
                        ).wait()
                        pltpu.make_async_copy(
                            xb_hbm.at[idx_v.at[s, 1]], rows_b.at[s], sem_b[s]
                        ).wait()

            @pl.when(c_next < c1)
            def _():
                for s in range(2):
                    @pl.when(nslot == s)
                    def _():
                        pltpu.async_copy(idx_hbm.at[c_next], idx_v.at[s], sem_i[s])

            for s in range(2):
                @pl.when(slot == s)
                def _():
                    @pl.when((i >= 2) & (c_prev2 < c1))
                    def _():
                        pltpu.make_async_copy(
                            rows_a.at[s],
                            g_hbm.at[pl.ds((c_prev2 - c0) * CHUNK, CHUNK)],
                            sem_w[s],
                        ).wait()

                    @pl.when(c_cur < c1)
                    def _():
                        pltpu.make_async_copy(
                            idx_hbm.at[c_cur], idx_v.at[s], sem_i[s]
                        ).wait()
                        pltpu.async_copy(
                            xa_hbm.at[idx_v.at[s, 0]], rows_a.at[s], sem_a[s]
                        )
                        pltpu.async_copy(
                            xb_hbm.at[idx_v.at[s, 1]], rows_b.at[s], sem_b[s]
                        )

            @pl.when((i >= 1) & (c_prev < c1))
            def _():
                for s in range(2):
                    @pl.when(nslot == s)
                    def _():
                        def row_add(r, cc):
                            for jj in range(D // 16):
                                sl = pl.ds(jj * 16, 16)
                                rows_a[s, r, sl] = rows_a[s, r, sl] + rows_b[s, r, sl]
                            return cc

                        lax.fori_loop(0, CHUNK, row_add, 0)
                        pltpu.async_copy(
                            rows_a.at[s],
                            g_hbm.at[pl.ds((c_prev - c0) * CHUNK, CHUNK)],
                            sem_w[s],
                        )

            return carry

        lax.fori_loop(0, SEG_ITERS + 2, body, 0)

    return k(xa, xb, idx2)


def _edge_body(g_ref, ea_ref, w1c_ref, w2_ref, b2_ref, out_ref):
    t = g_ref[...] + jnp.dot(
        ea_ref[...], w1c_ref[...], preferred_element_type=jnp.float32
    )
    h = _silu(t)
    out_ref[...] = (
        jnp.dot(h, w2_ref[...], preferred_element_type=jnp.float32)
        + b2_ref[...]
    )


def _edge_mlp(g, ea, w1c, w2, b2):
    blk = lambda i: (i, 0)
    whole = lambda i: (0, 0)
    return pl.pallas_call(
        _edge_body,
        grid=(NB_E,),
        in_specs=[
            pl.BlockSpec((BE, D), blk),
            pl.BlockSpec((BE, D), blk),
            pl.BlockSpec((D, D), whole),
            pl.BlockSpec((D, D), whole),
            pl.BlockSpec((1, D), whole),
        ],
        out_specs=pl.BlockSpec((BE, D), blk),
        out_shape=jax.ShapeDtypeStruct((E, D), jnp.float32),
    )(g, ea, w1c, w2, b2)


def _sc_scatter(enh, dst2d, init, seg):
    mesh = plsc.VectorSubcoreMesh(core_axis_name="c", subcore_axis_name="s")
    c0 = seg * SEG_CHUNKS
    c1 = c0 + SEG_CHUNKS

    @functools.partial(
        pl.kernel,
        out_type=jax.ShapeDtypeStruct((2, N, D), jnp.float32),
        mesh=mesh,
        scratch_types=[
            pltpu.VMEM((2, CHUNK), jnp.int32),
            pltpu.VMEM((2, CHUNK, D), jnp.float32),
            pltpu.VMEM_SHARED((N, D), jnp.float32),
        ] + [pltpu.SemaphoreType.DMA] * 6,
        name=f"sc_scatter_{seg}",
    )
    def k(en_hbm, dst_hbm, init_hbm, out_hbm, idx_v, rows_v, acc_sh,
          sem_i0, sem_i1, sem_r0, sem_r1, sem_s0, sem_s1):
        cid = lax.axis_index("c")
        sid = lax.axis_index("s")
        wid = sid * 2 + cid
        r0 = pl.multiple_of(sid * ROWS_PER_TILE, 8)
        sem_i = [sem_i0, sem_i1]
        sem_r = [sem_r0, sem_r1]
        sem_s = [sem_s0, sem_s1]

        pltpu.async_copy(dst_hbm.at[c0 + wid], idx_v.at[0], sem_i0)
        pltpu.async_copy(en_hbm.at[pl.ds(wid * CHUNK, CHUNK)], rows_v.at[0], sem_r0)

        pltpu.sync_copy(
            init_hbm.at[cid, pl.ds(r0, ROWS_PER_TILE)],
            acc_sh.at[pl.ds(r0, ROWS_PER_TILE)],
        )

        @pl.when(sid == 0)
        def _():
            pltpu.sync_copy(
                init_hbm.at[cid, pl.ds(TAIL_BASE, TAIL_ROWS)],
                acc_sh.at[pl.ds(TAIL_BASE, TAIL_ROWS)],
            )

        plsc.subcore_barrier()

        def body(i, carry):
            slot = lax.rem(i, 2)
            nslot = lax.rem(i + 1, 2)
            c_prev = c0 + wid + NWORKERS * (i - 1)
            c_cur = c0 + wid + NWORKERS * i
            c_next = c0 + wid + NWORKERS * (i + 1)

            @pl.when((i >= 1) & (c_prev < c1))
            def _():
                for s in range(2):
                    @pl.when(nslot == s)
                    def _():
                        pltpu.make_async_copy(
                            rows_v.at[s], acc_sh.at[idx_v.at[s]], sem_s[s]
                        ).wait()

            @pl.when(c_next < c1)
            def _():
                for s in range(2):
                    @pl.when(nslot == s)
                    def _():
                        pltpu.async_copy(dst_hbm.at[c_next], idx_v.at[s], sem_i[s])
                        pltpu.async_copy(
                            en_hbm.at[pl.ds((c_next - c0) * CHUNK, CHUNK)],
                            rows_v.at[s], sem_r[s],
                        )

            @pl.when(c_cur < c1)
            def _():
                for s in range(2):
                    @pl.when(slot == s)
                    def _():
                        pltpu.make_async_copy(
                            dst_hbm.at[c_cur], idx_v.at[s], sem_i[s]
                        ).wait()
                        pltpu.make_async_copy(
                            en_hbm.at[pl.ds((c_cur - c0) * CHUNK, CHUNK)],
                            rows_v.at[s], sem_r[s],
                        ).wait()
                        pltpu.async_copy(
                            rows_v.at[s], acc_sh.at[idx_v.at[s]], sem_s[s],
                            add=True,
                        )

            return carry

        lax.fori_loop(0, SEG_ITERS + 1, body, 0)
        plsc.subcore_barrier()
        pltpu.sync_copy(
            acc_sh.at[pl.ds(r0, ROWS_PER_TILE)],
            out_hbm.at[cid, pl.ds(r0, ROWS_PER_TILE)],
        )

        @pl.when(sid == 0)
        def _():
            pltpu.sync_copy(
                acc_sh.at[pl.ds(TAIL_BASE, TAIL_ROWS)],
                out_hbm.at[cid, pl.ds(TAIL_BASE, TAIL_ROWS)],
            )

    return k(enh, dst2d, init)


def _node_body(x_ref, p0_ref, p1_ref, b_ref, upad_ref,
               n1a_ref, n1b_ref, n1c_ref, nb1_ref, nw2_ref, nb2_ref,
               g1a_ref, g1b_ref, gb1_ref, gw2_ref, gb2_ref,
               xn_ref, uout_ref, sums_ref, cnt_ref):
    i = pl.program_id(0)

    @pl.when(i == 0)
    def _():
        sums_ref[...] = jnp.zeros((D, D), jnp.float32)
        cnt_ref[...] = jnp.zeros((D, D), jnp.float32)

    oh = (b_ref[...] == lax.broadcasted_iota(jnp.int32, (BN, D), 1)).astype(
        jnp.float32
    )
    ug = jnp.dot(upad_ref[...], n1c_ref[...], preferred_element_type=jnp.float32)
    msgs = (p0_ref[...] + p1_ref[...]) * INV_AVG_ADJ
    pre = (
        jnp.dot(x_ref[...], n1a_ref[...], preferred_element_type=jnp.float32)
        + jnp.dot(msgs, n1b_ref[...], preferred_element_type=jnp.float32)
        + jnp.dot(oh, ug, preferred_element_type=jnp.float32)
        + nb1_ref[...]
    )
    xn = (
        jnp.dot(_silu(pre), nw2_ref[...], preferred_element_type=jnp.float32)
        + nb2_ref[...]
    )
    xn_ref[...] = xn

    dims = (((0,), (0,)), ((), ()))
    sums_ref[...] += lax.dot_general(
        oh, xn, dims, preferred_element_type=jnp.float32
    )
    cnt_ref[...] += lax.dot_general(
        oh, jnp.ones((BN, D), jnp.float32), dims, preferred_element_type=jnp.float32
    )

    @pl.when(i == NB_N - 1)
    def _():
        mean = sums_ref[...] / jnp.maximum(cnt_ref[...], 1.0)
        t = (
            jnp.dot(upad_ref[...], g1a_ref[...], preferred_element_type=jnp.float32)
            + jnp.dot(mean, g1b_ref[...], preferred_element_type=jnp.float32)
            + gb1_ref[...]
        )
        uout_ref[...] = (
            jnp.dot(_silu(t), gw2_ref[...], preferred_element_type=jnp.float32)
            + gb2_ref[...]
        )


def _node_global(x, p0, p1, batch2d, upad,
                 n1a, n1b, n1c, nb1, nw2, nb2,
                 g1a, g1b, gb1, gw2, gb2):
    whole = lambda i: (0, 0)
    blk = lambda i: (i, 0)
    return pl.pallas_call(
        _node_body,
        grid=(NB_N,),
        in_specs=[
            pl.BlockSpec((BN, D), blk),
            pl.BlockSpec((BN, D), blk),
            pl.BlockSpec((BN, D), blk),
            pl.BlockSpec((BN, 1), blk),
            pl.BlockSpec((D, D), whole),
            pl.BlockSpec((D, D), whole),
            pl.BlockSpec((D, D), whole),
            pl.BlockSpec((D, D), whole),
            pl.BlockSpec((1, D), whole),
            pl.BlockSpec((D, D), whole),
            pl.BlockSpec((1, D), whole),
            pl.BlockSpec((D, D), whole),
            pl.BlockSpec((D, D), whole),
            pl.BlockSpec((1, D), whole),
            pl.BlockSpec((D, D), whole),
            pl.BlockSpec((1, D), whole),
        ],
        out_specs=[
            pl.BlockSpec((BN, D), blk),
            pl.BlockSpec((D, D), whole),
        ],
        out_shape=[
            jax.ShapeDtypeStruct((N, D), jnp.float32),
            jax.ShapeDtypeStruct((D, D), jnp.float32),
        ],
        scratch_shapes=[
            pltpu.VMEM((D, D), jnp.float32),
            pltpu.VMEM((D, D), jnp.float32),
        ],
    )(x, p0, p1, batch2d, upad,
      n1a, n1b, n1c, nb1, nw2, nb2,
      g1a, g1b, gb1, gw2, gb2)


def kernel(x, edge_index, edge_attr, u, batch,
           e_w1, e_b1, e_w2, e_b2,
           n_w1, n_b1, n_w2, n_b2,
           g_w1, g_b1, g_w2, g_b2):
    src2d = edge_index[0].reshape(NCHUNKS, CHUNK)
    dst2d = edge_index[1].reshape(NCHUNKS, CHUNK)
    idx2 = jnp.stack([src2d, dst2d], axis=1)
    w1a, w1b, w1c = e_w1[:D], e_w1[D:2 * D], e_w1[2 * D:]
    n1a, n1b, n1c = n_w1[:D], n_w1[D:2 * D], n_w1[2 * D:]
    g1a, g1b = g_w1[:D], g_w1[D:]
    upad = jnp.zeros((D, D), jnp.float32).at[:G].set(u)
    batch2d = batch.reshape(N, 1)

    xa, xb = _proj(x, w1a, w1b, e_b1.reshape(1, D))
    g = _sc_gather(xa, xb, idx2, 0)
    en = _edge_mlp(g, edge_attr, w1c, e_w2, e_b2.reshape(1, D))
    zeros = jnp.zeros((2, N, D), jnp.float32)
    partials = _sc_scatter(en, dst2d, zeros, 0)
    x_new, uout = _node_global(
        x, partials[0], partials[1], batch2d, upad,
        n1a, n1b, n1c, n_b1.reshape(1, D), n_w2, n_b2.reshape(1, D),
        g1a, g1b, g_b1.reshape(1, D), g_w2, g_b2.reshape(1, D),
    )
    return (x_new, en, uout[:G])


# speedup vs baseline: 1.2458x; 1.0147x over previous
import functools

import jax
import jax.numpy as jnp
from jax import lax
from jax.experimental import pallas as pl
from jax.experimental.pallas import tpu as pltpu
from jax.experimental.pallas import tpu_sc as plsc

N = 10000
E = 160000
D = 128
G = 100
INV_AVG_ADJ = 1.0 / 16.0

NB_N = 5
BN = N // NB_N
NB_E = 20
BE = E // NB_E
CHUNK = 128
NCHUNKS = E // CHUNK
NWORKERS = 32
NSEG = 1
SEG_CHUNKS = NCHUNKS // NSEG
SEG_E = E // NSEG
SEG_BLOCKS = SEG_E // 1600
SEG_ITERS = -(-SEG_CHUNKS // NWORKERS)
ROWS_PER_TILE = 624
TAIL_ROWS = N - 16 * ROWS_PER_TILE
TAIL_BASE = 16 * ROWS_PER_TILE


def _silu(t):
    return t * jax.nn.sigmoid(t)


def _proj_body(x_ref, w1a_ref, w1b_ref, b1_ref, xa_ref, xb_ref):
    xblk = x_ref[...]
    xa_ref[...] = (
        jnp.dot(xblk, w1a_ref[...], preferred_element_type=jnp.float32)
        + b1_ref[...]
    )
    xb_ref[...] = jnp.dot(xblk, w1b_ref[...], preferred_element_type=jnp.float32)


def _proj(x, w1a, w1b, b1):
    bp = 2000
    return pl.pallas_call(
        _proj_body,
        grid=(N // bp,),
        in_specs=[
            pl.BlockSpec((bp, D), lambda i: (i, 0)),
            pl.BlockSpec((D, D), lambda i: (0, 0)),
            pl.BlockSpec((D, D), lambda i: (0, 0)),
            pl.BlockSpec((1, D), lambda i: (0, 0)),
        ],
        out_specs=[
            pl.BlockSpec((bp, D), lambda i: (i, 0)),
            pl.BlockSpec((bp, D), lambda i: (i, 0)),
        ],
        out_shape=[jax.ShapeDtypeStruct((N, D), jnp.float32)] * 2,
    )(x, w1a, w1b, b1)


def _sc_gather(xa, xb, idx2, seg):
    mesh = plsc.VectorSubcoreMesh(core_axis_name="c", subcore_axis_name="s")
    c0 = seg * SEG_CHUNKS
    c1 = c0 + SEG_CHUNKS

    @functools.partial(
        pl.kernel,
        out_type=jax.ShapeDtypeStruct((SEG_E, D), jnp.float32),
        mesh=mesh,
        scratch_types=[
            pltpu.VMEM((2, 2, CHUNK), jnp.int32),
            pltpu.VMEM((2, CHUNK, D), jnp.float32),
            pltpu.VMEM((2, CHUNK, D), jnp.float32),
        ] + [pltpu.SemaphoreType.DMA] * 8,
        name=f"sc_gather_{seg}",
    )
    def k(xa_hbm, xb_hbm, idx_hbm, g_hbm,
          idx_v, rows_a, rows_b,
          sem_i0, sem_i1, sem_a0, sem_a1, sem_b0, sem_b1, sem_w0, sem_w1):
        wid = lax.axis_index("s") * 2 + lax.axis_index("c")
        sem_i = [sem_i0, sem_i1]
        sem_a = [sem_a0, sem_a1]
        sem_b = [sem_b0, sem_b1]
        sem_w = [sem_w0, sem_w1]

        pltpu.async_copy(idx_hbm.at[c0 + wid], idx_v.at[0], sem_i0)

        def body(i, carry):
            slot = lax.rem(i, 2)
            nslot = lax.rem(i + 1, 2)
            c_prev = c0 + wid + NWORKERS * (i - 1)
            c_cur = c0 + wid + NWORKERS * i
            c_next = c0 + wid + NWORKERS * (i + 1)
            c_prev2 = c0 + wid + NWORKERS * (i - 2)

            @pl.when((i >= 1) & (c_prev < c1))
            def _():
                for s in range(2):
                    @pl.when(nslot == s)
                    def _():
                        pltpu.make_async_copy(
                            xa_hbm.at[idx_v.at[s, 0]], rows_a.at[s], sem_a[s]
                        ).wait()
                        pltpu.make_async_copy(
                            xb_hbm.at[idx_v.at[s, 1]], rows_b.at[s], sem_b[s]
                        ).wait()

            @pl.when(c_next < c1)
            def _():
                for s in range(2):
                    @pl.when(nslot == s)
                    def _():
                        pltpu.async_copy(idx_hbm.at[c_next], idx_v.at[s], sem_i[s])

            for s in range(2):
                @pl.when(slot == s)
                def _():
                    @pl.when((i >= 2) & (c_prev2 < c1))
                    def _():
                        pltpu.make_async_copy(
                            rows_a.at[s],
                            g_hbm.at[pl.ds((c_prev2 - c0) * CHUNK, CHUNK)],
                            sem_w[s],
                        ).wait()

                    @pl.when(c_cur < c1)
                    def _():
                        pltpu.make_async_copy(
                            idx_hbm.at[c_cur], idx_v.at[s], sem_i[s]
                        ).wait()
                        pltpu.async_copy(
                            xa_hbm.at[idx_v.at[s, 0]], rows_a.at[s], sem_a[s]
                        )
                        pltpu.async_copy(
                            xb_hbm.at[idx_v.at[s, 1]], rows_b.at[s], sem_b[s]
                        )

            @pl.when((i >= 1) & (c_prev < c1))
            def _():
                for s in range(2):
                    @pl.when(nslot == s)
                    def _():
                        def row_add(r, cc):
                            for jj in range(D // 16):
                                sl = pl.ds(jj * 16, 16)
                                rows_a[s, r, sl] = rows_a[s, r, sl] + rows_b[s, r, sl]
                            return cc

                        lax.fori_loop(0, CHUNK, row_add, 0)
                        pltpu.async_copy(
                            rows_a.at[s],
                            g_hbm.at[pl.ds((c_prev - c0) * CHUNK, CHUNK)],
                            sem_w[s],
                        )

            return carry

        lax.fori_loop(0, SEG_ITERS + 2, body, 0)

    return k(xa, xb, idx2)


def _edge_body(g_ref, ea_ref, w1c_ref, w2_ref, b2_ref, out_ref):
    t = g_ref[...] + jnp.dot(
        ea_ref[...], w1c_ref[...], preferred_element_type=jnp.float32
    )
    h = _silu(t)
    out_ref[...] = (
        jnp.dot(h, w2_ref[...], preferred_element_type=jnp.float32)
        + b2_ref[...]
    )


def _edge_mlp(g, ea, w1c, w2, b2):
    blk = lambda i: (i, 0)
    whole = lambda i: (0, 0)
    return pl.pallas_call(
        _edge_body,
        grid=(NB_E,),
        in_specs=[
            pl.BlockSpec((BE, D), blk),
            pl.BlockSpec((BE, D), blk),
            pl.BlockSpec((D, D), whole),
            pl.BlockSpec((D, D), whole),
            pl.BlockSpec((1, D), whole),
        ],
        out_specs=pl.BlockSpec((BE, D), blk),
        out_shape=jax.ShapeDtypeStruct((E, D), jnp.float32),
    )(g, ea, w1c, w2, b2)


def _sc_scatter(enh, dst2d, init, seg):
    mesh = plsc.VectorSubcoreMesh(core_axis_name="c", subcore_axis_name="s")
    c0 = seg * SEG_CHUNKS
    c1 = c0 + SEG_CHUNKS

    @functools.partial(
        pl.kernel,
        out_type=jax.ShapeDtypeStruct((2, N, D), jnp.float32),
        mesh=mesh,
        scratch_types=[
            pltpu.VMEM((2, CHUNK), jnp.int32),
            pltpu.VMEM((2, CHUNK, D), jnp.float32),
            pltpu.VMEM_SHARED((N, D), jnp.float32),
        ] + [pltpu.SemaphoreType.DMA] * 6,
        name=f"sc_scatter_{seg}",
    )
    def k(en_hbm, dst_hbm, init_hbm, out_hbm, idx_v, rows_v, acc_sh,
          sem_i0, sem_i1, sem_r0, sem_r1, sem_s0, sem_s1):
        cid = lax.axis_index("c")
        sid = lax.axis_index("s")
        wid = sid * 2 + cid
        r0 = pl.multiple_of(sid * ROWS_PER_TILE, 8)
        sem_i = [sem_i0, sem_i1]
        sem_r = [sem_r0, sem_r1]
        sem_s = [sem_s0, sem_s1]

        pltpu.async_copy(dst_hbm.at[c0 + wid], idx_v.at[0], sem_i0)
        pltpu.async_copy(en_hbm.at[pl.ds(wid * CHUNK, CHUNK)], rows_v.at[0], sem_r0)

        pltpu.sync_copy(
            init_hbm.at[cid, pl.ds(r0, ROWS_PER_TILE)],
            acc_sh.at[pl.ds(r0, ROWS_PER_TILE)],
        )

        @pl.when(sid == 0)
        def _():
            pltpu.sync_copy(
                init_hbm.at[cid, pl.ds(TAIL_BASE, TAIL_ROWS)],
                acc_sh.at[pl.ds(TAIL_BASE, TAIL_ROWS)],
            )

        plsc.subcore_barrier()

        def body(i, carry):
            slot = lax.rem(i, 2)
            nslot = lax.rem(i + 1, 2)
            c_prev = c0 + wid + NWORKERS * (i - 1)
            c_cur = c0 + wid + NWORKERS * i
            c_next = c0 + wid + NWORKERS * (i + 1)

            @pl.when((i >= 1) & (c_prev < c1))
            def _():
                for s in range(2):
                    @pl.when(nslot == s)
                    def _():
                        pltpu.make_async_copy(
                            rows_v.at[s], acc_sh.at[idx_v.at[s]], sem_s[s]
                        ).wait()

            @pl.when(c_next < c1)
            def _():
                for s in range(2):
                    @pl.when(nslot == s)
                    def _():
                        pltpu.async_copy(dst_hbm.at[c_next], idx_v.at[s], sem_i[s])
                        pltpu.async_copy(
                            en_hbm.at[pl.ds((c_next - c0) * CHUNK, CHUNK)],
                            rows_v.at[s], sem_r[s],
                        )

            @pl.when(c_cur < c1)
            def _():
                for s in range(2):
                    @pl.when(slot == s)
                    def _():
                        pltpu.make_async_copy(
                            dst_hbm.at[c_cur], idx_v.at[s], sem_i[s]
                        ).wait()
                        pltpu.make_async_copy(
                            en_hbm.at[pl.ds((c_cur - c0) * CHUNK, CHUNK)],
                            rows_v.at[s], sem_r[s],
                        ).wait()
                        pltpu.async_copy(
                            rows_v.at[s], acc_sh.at[idx_v.at[s]], sem_s[s],
                            add=True,
                        )

            return carry

        lax.fori_loop(0, SEG_ITERS + 1, body, 0)
        plsc.subcore_barrier()
        pltpu.sync_copy(
            acc_sh.at[pl.ds(r0, ROWS_PER_TILE)],
            out_hbm.at[cid, pl.ds(r0, ROWS_PER_TILE)],
        )

        @pl.when(sid == 0)
        def _():
            pltpu.sync_copy(
                acc_sh.at[pl.ds(TAIL_BASE, TAIL_ROWS)],
                out_hbm.at[cid, pl.ds(TAIL_BASE, TAIL_ROWS)],
            )

    return k(enh, dst2d, init)


def _node_body(x_ref, p0_ref, p1_ref, b_ref, upad_ref,
               n1a_ref, n1b_ref, n1c_ref, nb1_ref, nw2_ref, nb2_ref,
               g1a_ref, g1b_ref, gb1_ref, gw2_ref, gb2_ref,
               xn_ref, uout_ref, sums_ref, cnt_ref):
    i = pl.program_id(0)

    @pl.when(i == 0)
    def _():
        sums_ref[...] = jnp.zeros((D, D), jnp.float32)
        cnt_ref[...] = jnp.zeros((D, D), jnp.float32)

    oh = (b_ref[...] == lax.broadcasted_iota(jnp.int32, (BN, D), 1)).astype(
        jnp.float32
    )
    ug = jnp.dot(upad_ref[...], n1c_ref[...], preferred_element_type=jnp.float32)
    msgs = (p0_ref[...] + p1_ref[...]) * INV_AVG_ADJ
    pre = (
        jnp.dot(x_ref[...], n1a_ref[...], preferred_element_type=jnp.float32)
        + jnp.dot(msgs, n1b_ref[...], preferred_element_type=jnp.float32)
        + jnp.dot(oh, ug, preferred_element_type=jnp.float32)
        + nb1_ref[...]
    )
    xn = (
        jnp.dot(_silu(pre), nw2_ref[...], preferred_element_type=jnp.float32)
        + nb2_ref[...]
    )
    xn_ref[...] = xn

    dims = (((0,), (0,)), ((), ()))
    sums_ref[...] += lax.dot_general(
        oh, xn, dims, preferred_element_type=jnp.float32
    )
    cnt_ref[...] += lax.dot_general(
        oh, jnp.ones((BN, D), jnp.float32), dims, preferred_element_type=jnp.float32
    )

    @pl.when(i == NB_N - 1)
    def _():
        mean = sums_ref[...] / jnp.maximum(cnt_ref[...], 1.0)
        t = (
            jnp.dot(upad_ref[...], g1a_ref[...], preferred_element_type=jnp.float32)
            + jnp.dot(mean, g1b_ref[...], preferred_element_type=jnp.float32)
            + gb1_ref[...]
        )
        uout_ref[...] = (
            jnp.dot(_silu(t), gw2_ref[...], preferred_element_type=jnp.float32)
            + gb2_ref[...]
        )


def _node_global(x, p0, p1, batch2d, upad,
                 n1a, n1b, n1c, nb1, nw2, nb2,
                 g1a, g1b, gb1, gw2, gb2):
    whole = lambda i: (0, 0)
    blk = lambda i: (i, 0)
    return pl.pallas_call(
        _node_body,
        grid=(NB_N,),
        in_specs=[
            pl.BlockSpec((BN, D), blk),
            pl.BlockSpec((BN, D), blk),
            pl.BlockSpec((BN, D), blk),
            pl.BlockSpec((BN, 1), blk),
            pl.BlockSpec((D, D), whole),
            pl.BlockSpec((D, D), whole),
            pl.BlockSpec((D, D), whole),
            pl.BlockSpec((D, D), whole),
            pl.BlockSpec((1, D), whole),
            pl.BlockSpec((D, D), whole),
            pl.BlockSpec((1, D), whole),
            pl.BlockSpec((D, D), whole),
            pl.BlockSpec((D, D), whole),
            pl.BlockSpec((1, D), whole),
            pl.BlockSpec((D, D), whole),
            pl.BlockSpec((1, D), whole),
        ],
        out_specs=[
            pl.BlockSpec((BN, D), blk),
            pl.BlockSpec((D, D), whole),
        ],
        out_shape=[
            jax.ShapeDtypeStruct((N, D), jnp.float32),
            jax.ShapeDtypeStruct((D, D), jnp.float32),
        ],
        scratch_shapes=[
            pltpu.VMEM((D, D), jnp.float32),
            pltpu.VMEM((D, D), jnp.float32),
        ],
    )(x, p0, p1, batch2d, upad,
      n1a, n1b, n1c, nb1, nw2, nb2,
      g1a, g1b, gb1, gw2, gb2)


def kernel(x, edge_index, edge_attr, u, batch,
           e_w1, e_b1, e_w2, e_b2,
           n_w1, n_b1, n_w2, n_b2,
           g_w1, g_b1, g_w2, g_b2):
    src2d = edge_index[0].reshape(NCHUNKS, CHUNK)
    dst2d = edge_index[1].reshape(NCHUNKS, CHUNK)
    idx2 = jnp.stack([src2d, dst2d], axis=1)
    w1a, w1b, w1c = e_w1[:D], e_w1[D:2 * D], e_w1[2 * D:]
    n1a, n1b, n1c = n_w1[:D], n_w1[D:2 * D], n_w1[2 * D:]
    g1a, g1b = g_w1[:D], g_w1[D:]
    upad = jnp.zeros((D, D), jnp.float32).at[:G].set(u)
    batch2d = batch.reshape(N, 1)

    xa, xb = _proj(x, w1a, w1b, e_b1.reshape(1, D))
    g = _sc_gather(xa, xb, idx2, 0)
    en = _edge_mlp(g, edge_attr, w1c, e_w2, e_b2.reshape(1, D))
    zeros = jnp.zeros((2, N, D), jnp.float32)
    partials = _sc_scatter(en, dst2d, zeros, 0)
    x_new, uout = _node_global(
        x, partials[0], partials[1], batch2d, upad,
        n1a, n1b, n1c, n_b1.reshape(1, D), n_w2, n_b2.reshape(1, D),
        g1a, g1b, g_b1.reshape(1, D), g_w2, g_b2.reshape(1, D),
    )
    return (x_new, en, uout[:G])


# speedup vs baseline: 1.2513x; 1.0044x over previous
import functools

import jax
import jax.numpy as jnp
from jax import lax
from jax.experimental import pallas as pl
from jax.experimental.pallas import tpu as pltpu
from jax.experimental.pallas import tpu_sc as plsc

N = 10000
E = 160000
D = 128
G = 100
INV_AVG_ADJ = 1.0 / 16.0

NB_N = 5
BN = N // NB_N
NB_E = 16
BE = E // NB_E
CHUNK = 128
NCHUNKS = E // CHUNK
NWORKERS = 32
NSEG = 1
SEG_CHUNKS = NCHUNKS // NSEG
SEG_E = E // NSEG
SEG_BLOCKS = SEG_E // 1600
SEG_ITERS = -(-SEG_CHUNKS // NWORKERS)
ROWS_PER_TILE = 624
TAIL_ROWS = N - 16 * ROWS_PER_TILE
TAIL_BASE = 16 * ROWS_PER_TILE


def _silu(t):
    return t * jax.nn.sigmoid(t)


def _proj_body(x_ref, w1a_ref, w1b_ref, b1_ref, xa_ref, xb_ref):
    xblk = x_ref[...]
    xa_ref[...] = (
        jnp.dot(xblk, w1a_ref[...], preferred_element_type=jnp.float32)
        + b1_ref[...]
    )
    xb_ref[...] = jnp.dot(xblk, w1b_ref[...], preferred_element_type=jnp.float32)


def _proj(x, w1a, w1b, b1):
    bp = 2000
    return pl.pallas_call(
        _proj_body,
        grid=(N // bp,),
        in_specs=[
            pl.BlockSpec((bp, D), lambda i: (i, 0)),
            pl.BlockSpec((D, D), lambda i: (0, 0)),
            pl.BlockSpec((D, D), lambda i: (0, 0)),
            pl.BlockSpec((1, D), lambda i: (0, 0)),
        ],
        out_specs=[
            pl.BlockSpec((bp, D), lambda i: (i, 0)),
            pl.BlockSpec((bp, D), lambda i: (i, 0)),
        ],
        out_shape=[jax.ShapeDtypeStruct((N, D), jnp.float32)] * 2,
    )(x, w1a, w1b, b1)


def _sc_gather(xa, xb, idx2, seg):
    mesh = plsc.VectorSubcoreMesh(core_axis_name="c", subcore_axis_name="s")
    c0 = seg * SEG_CHUNKS
    c1 = c0 + SEG_CHUNKS

    @functools.partial(
        pl.kernel,
        out_type=jax.ShapeDtypeStruct((SEG_E, D), jnp.float32),
        mesh=mesh,
        scratch_types=[
            pltpu.VMEM((2, 2, CHUNK), jnp.int32),
            pltpu.VMEM((2, CHUNK, D), jnp.float32),
            pltpu.VMEM((2, CHUNK, D), jnp.float32),
        ] + [pltpu.SemaphoreType.DMA] * 8,
        name=f"sc_gather_{seg}",
    )
    def k(xa_hbm, xb_hbm, idx_hbm, g_hbm,
          idx_v, rows_a, rows_b,
          sem_i0, sem_i1, sem_a0, sem_a1, sem_b0, sem_b1, sem_w0, sem_w1):
        wid = lax.axis_index("s") * 2 + lax.axis_index("c")
        sem_i = [sem_i0, sem_i1]
        sem_a = [sem_a0, sem_a1]
        sem_b = [sem_b0, sem_b1]
        sem_w = [sem_w0, sem_w1]

        pltpu.async_copy(idx_hbm.at[c0 + wid], idx_v.at[0], sem_i0)

        def body(i, carry):
            slot = lax.rem(i, 2)
            nslot = lax.rem(i + 1, 2)
            c_prev = c0 + wid + NWORKERS * (i - 1)
            c_cur = c0 + wid + NWORKERS * i
            c_next = c0 + wid + NWORKERS * (i + 1)
            c_prev2 = c0 + wid + NWORKERS * (i - 2)

            @pl.when((i >= 1) & (c_prev < c1))
            def _():
                for s in range(2):
                    @pl.when(nslot == s)
                    def _():
                        pltpu.make_async_copy(
                            xa_hbm.at[idx_v.at[s, 0]], rows_a.at[s], sem_a[s]
                        ).wait()
                        pltpu.make_async_copy(
                            xb_hbm.at[idx_v.at[s, 1]], rows_b.at[s], sem_b[s]
                        ).wait()

            @pl.when(c_next < c1)
            def _():
                for s in range(2):
                    @pl.when(nslot == s)
                    def _():
                        pltpu.async_copy(idx_hbm.at[c_next], idx_v.at[s], sem_i[s])

            for s in range(2):
                @pl.when(slot == s)
                def _():
                    @pl.when((i >= 2) & (c_prev2 < c1))
                    def _():
                        pltpu.make_async_copy(
                            rows_a.at[s],
                            g_hbm.at[pl.ds((c_prev2 - c0) * CHUNK, CHUNK)],
                            sem_w[s],
                        ).wait()

                    @pl.when(c_cur < c1)
                    def _():
                        pltpu.make_async_copy(
                            idx_hbm.at[c_cur], idx_v.at[s], sem_i[s]
                        ).wait()
                        pltpu.async_copy(
                            xa_hbm.at[idx_v.at[s, 0]], rows_a.at[s], sem_a[s]
                        )
                        pltpu.async_copy(
                            xb_hbm.at[idx_v.at[s, 1]], rows_b.at[s], sem_b[s]
                        )

            @pl.when((i >= 1) & (c_prev < c1))
            def _():
                for s in range(2):
                    @pl.when(nslot == s)
                    def _():
                        def row_add(r, cc):
                            for jj in range(D // 16):
                                sl = pl.ds(jj * 16, 16)
                                rows_a[s, r, sl] = rows_a[s, r, sl] + rows_b[s, r, sl]
                            return cc

                        lax.fori_loop(0, CHUNK, row_add, 0)
                        pltpu.async_copy(
                            rows_a.at[s],
                            g_hbm.at[pl.ds((c_prev - c0) * CHUNK, CHUNK)],
                            sem_w[s],
                        )

            return carry

        lax.fori_loop(0, SEG_ITERS + 2, body, 0)

    return k(xa, xb, idx2)


def _edge_body(g_ref, ea_ref, w1c_ref, w2_ref, b2_ref, out_ref):
    t = g_ref[...] + jnp.dot(
        ea_ref[...], w1c_ref[...], preferred_element_type=jnp.float32
    )
    h = _silu(t)
    out_ref[...] = (
        jnp.dot(h, w2_ref[...], preferred_element_type=jnp.float32)
        + b2_ref[...]
    )


def _edge_mlp(g, ea, w1c, w2, b2):
    blk = lambda i: (i, 0)
    whole = lambda i: (0, 0)
    return pl.pallas_call(
        _edge_body,
        grid=(NB_E,),
        in_specs=[
            pl.BlockSpec((BE, D), blk),
            pl.BlockSpec((BE, D), blk),
            pl.BlockSpec((D, D), whole),
            pl.BlockSpec((D, D), whole),
            pl.BlockSpec((1, D), whole),
        ],
        out_specs=pl.BlockSpec((BE, D), blk),
        out_shape=jax.ShapeDtypeStruct((E, D), jnp.float32),
    )(g, ea, w1c, w2, b2)


def _sc_scatter(enh, dst2d, init, seg):
    mesh = plsc.VectorSubcoreMesh(core_axis_name="c", subcore_axis_name="s")
    c0 = seg * SEG_CHUNKS
    c1 = c0 + SEG_CHUNKS

    @functools.partial(
        pl.kernel,
        out_type=jax.ShapeDtypeStruct((2, N, D), jnp.float32),
        mesh=mesh,
        scratch_types=[
            pltpu.VMEM((2, CHUNK), jnp.int32),
            pltpu.VMEM((2, CHUNK, D), jnp.float32),
            pltpu.VMEM_SHARED((N, D), jnp.float32),
        ] + [pltpu.SemaphoreType.DMA] * 6,
        name=f"sc_scatter_{seg}",
    )
    def k(en_hbm, dst_hbm, init_hbm, out_hbm, idx_v, rows_v, acc_sh,
          sem_i0, sem_i1, sem_r0, sem_r1, sem_s0, sem_s1):
        cid = lax.axis_index("c")
        sid = lax.axis_index("s")
        wid = sid * 2 + cid
        r0 = pl.multiple_of(sid * ROWS_PER_TILE, 8)
        sem_i = [sem_i0, sem_i1]
        sem_r = [sem_r0, sem_r1]
        sem_s = [sem_s0, sem_s1]

        pltpu.async_copy(dst_hbm.at[c0 + wid], idx_v.at[0], sem_i0)
        pltpu.async_copy(en_hbm.at[pl.ds(wid * CHUNK, CHUNK)], rows_v.at[0], sem_r0)

        pltpu.sync_copy(
            init_hbm.at[cid, pl.ds(r0, ROWS_PER_TILE)],
            acc_sh.at[pl.ds(r0, ROWS_PER_TILE)],
        )

        @pl.when(sid == 0)
        def _():
            pltpu.sync_copy(
                init_hbm.at[cid, pl.ds(TAIL_BASE, TAIL_ROWS)],
                acc_sh.at[pl.ds(TAIL_BASE, TAIL_ROWS)],
            )

        plsc.subcore_barrier()

        def body(i, carry):
            slot = lax.rem(i, 2)
            nslot = lax.rem(i + 1, 2)
            c_prev = c0 + wid + NWORKERS * (i - 1)
            c_cur = c0 + wid + NWORKERS * i
            c_next = c0 + wid + NWORKERS * (i + 1)

            @pl.when((i >= 1) & (c_prev < c1))
            def _():
                for s in range(2):
                    @pl.when(nslot == s)
                    def _():
                        pltpu.make_async_copy(
                            rows_v.at[s], acc_sh.at[idx_v.at[s]], sem_s[s]
                        ).wait()

            @pl.when(c_next < c1)
            def _():
                for s in range(2):
                    @pl.when(nslot == s)
                    def _():
                        pltpu.async_copy(dst_hbm.at[c_next], idx_v.at[s], sem_i[s])
                        pltpu.async_copy(
                            en_hbm.at[pl.ds((c_next - c0) * CHUNK, CHUNK)],
                            rows_v.at[s], sem_r[s],
                        )

            @pl.when(c_cur < c1)
            def _():
                for s in range(2):
                    @pl.when(slot == s)
                    def _():
                        pltpu.make_async_copy(
                            dst_hbm.at[c_cur], idx_v.at[s], sem_i[s]
                        ).wait()
                        pltpu.make_async_copy(
                            en_hbm.at[pl.ds((c_cur - c0) * CHUNK, CHUNK)],
                            rows_v.at[s], sem_r[s],
                        ).wait()
                        pltpu.async_copy(
                            rows_v.at[s], acc_sh.at[idx_v.at[s]], sem_s[s],
                            add=True,
                        )

            return carry

        lax.fori_loop(0, SEG_ITERS + 1, body, 0)
        plsc.subcore_barrier()
        pltpu.sync_copy(
            acc_sh.at[pl.ds(r0, ROWS_PER_TILE)],
            out_hbm.at[cid, pl.ds(r0, ROWS_PER_TILE)],
        )

        @pl.when(sid == 0)
        def _():
            pltpu.sync_copy(
                acc_sh.at[pl.ds(TAIL_BASE, TAIL_ROWS)],
                out_hbm.at[cid, pl.ds(TAIL_BASE, TAIL_ROWS)],
            )

    return k(enh, dst2d, init)


def _node_body(x_ref, p0_ref, p1_ref, b_ref, upad_ref,
               n1a_ref, n1b_ref, n1c_ref, nb1_ref, nw2_ref, nb2_ref,
               g1a_ref, g1b_ref, gb1_ref, gw2_ref, gb2_ref,
               xn_ref, uout_ref, sums_ref, cnt_ref):
    i = pl.program_id(0)

    @pl.when(i == 0)
    def _():
        sums_ref[...] = jnp.zeros((D, D), jnp.float32)
        cnt_ref[...] = jnp.zeros((D, D), jnp.float32)

    oh = (b_ref[...] == lax.broadcasted_iota(jnp.int32, (BN, D), 1)).astype(
        jnp.float32
    )
    ug = jnp.dot(upad_ref[...], n1c_ref[...], preferred_element_type=jnp.float32)
    msgs = (p0_ref[...] + p1_ref[...]) * INV_AVG_ADJ
    pre = (
        jnp.dot(x_ref[...], n1a_ref[...], preferred_element_type=jnp.float32)
        + jnp.dot(msgs, n1b_ref[...], preferred_element_type=jnp.float32)
        + jnp.dot(oh, ug, preferred_element_type=jnp.float32)
        + nb1_ref[...]
    )
    xn = (
        jnp.dot(_silu(pre), nw2_ref[...], preferred_element_type=jnp.float32)
        + nb2_ref[...]
    )
    xn_ref[...] = xn

    dims = (((0,), (0,)), ((), ()))
    sums_ref[...] += lax.dot_general(
        oh, xn, dims, preferred_element_type=jnp.float32
    )
    cnt_ref[...] += lax.dot_general(
        oh, jnp.ones((BN, D), jnp.float32), dims, preferred_element_type=jnp.float32
    )

    @pl.when(i == NB_N - 1)
    def _():
        mean = sums_ref[...] / jnp.maximum(cnt_ref[...], 1.0)
        t = (
            jnp.dot(upad_ref[...], g1a_ref[...], preferred_element_type=jnp.float32)
            + jnp.dot(mean, g1b_ref[...], preferred_element_type=jnp.float32)
            + gb1_ref[...]
        )
        uout_ref[...] = (
            jnp.dot(_silu(t), gw2_ref[...], preferred_element_type=jnp.float32)
            + gb2_ref[...]
        )


def _node_global(x, p0, p1, batch2d, upad,
                 n1a, n1b, n1c, nb1, nw2, nb2,
                 g1a, g1b, gb1, gw2, gb2):
    whole = lambda i: (0, 0)
    blk = lambda i: (i, 0)
    return pl.pallas_call(
        _node_body,
        grid=(NB_N,),
        in_specs=[
            pl.BlockSpec((BN, D), blk),
            pl.BlockSpec((BN, D), blk),
            pl.BlockSpec((BN, D), blk),
            pl.BlockSpec((BN, 1), blk),
            pl.BlockSpec((D, D), whole),
            pl.BlockSpec((D, D), whole),
            pl.BlockSpec((D, D), whole),
            pl.BlockSpec((D, D), whole),
            pl.BlockSpec((1, D), whole),
            pl.BlockSpec((D, D), whole),
            pl.BlockSpec((1, D), whole),
            pl.BlockSpec((D, D), whole),
            pl.BlockSpec((D, D), whole),
            pl.BlockSpec((1, D), whole),
            pl.BlockSpec((D, D), whole),
            pl.BlockSpec((1, D), whole),
        ],
        out_specs=[
            pl.BlockSpec((BN, D), blk),
            pl.BlockSpec((D, D), whole),
        ],
        out_shape=[
            jax.ShapeDtypeStruct((N, D), jnp.float32),
            jax.ShapeDtypeStruct((D, D), jnp.float32),
        ],
        scratch_shapes=[
            pltpu.VMEM((D, D), jnp.float32),
            pltpu.VMEM((D, D), jnp.float32),
        ],
    )(x, p0, p1, batch2d, upad,
      n1a, n1b, n1c, nb1, nw2, nb2,
      g1a, g1b, gb1, gw2, gb2)


def kernel(x, edge_index, edge_attr, u, batch,
           e_w1, e_b1, e_w2, e_b2,
           n_w1, n_b1, n_w2, n_b2,
           g_w1, g_b1, g_w2, g_b2):
    src2d = edge_index[0].reshape(NCHUNKS, CHUNK)
    dst2d = edge_index[1].reshape(NCHUNKS, CHUNK)
    idx2 = jnp.stack([src2d, dst2d], axis=1)
    w1a, w1b, w1c = e_w1[:D], e_w1[D:2 * D], e_w1[2 * D:]
    n1a, n1b, n1c = n_w1[:D], n_w1[D:2 * D], n_w1[2 * D:]
    g1a, g1b = g_w1[:D], g_w1[D:]
    upad = jnp.zeros((D, D), jnp.float32).at[:G].set(u)
    batch2d = batch.reshape(N, 1)

    xa, xb = _proj(x, w1a, w1b, e_b1.reshape(1, D))
    g = _sc_gather(xa, xb, idx2, 0)
    en = _edge_mlp(g, edge_attr, w1c, e_w2, e_b2.reshape(1, D))
    zeros = jnp.zeros((2, N, D), jnp.float32)
    partials = _sc_scatter(en, dst2d, zeros, 0)
    x_new, uout = _node_global(
        x, partials[0], partials[1], batch2d, upad,
        n1a, n1b, n1c, n_b1.reshape(1, D), n_w2, n_b2.reshape(1, D),
        g1a, g1b, g_b1.reshape(1, D), g_w2, g_b2.reshape(1, D),
    )
    return (x_new, en, uout[:G])
